# Initial kernel scaffold; baseline (speedup 1.0000x reference)
import functools

import jax
import jax.numpy as jnp
from jax import lax
from jax.experimental import pallas as pl
from jax.experimental.pallas import tpu as pltpu
from jax.experimental.pallas import tpu_sc as plsc

_N0, _N1, _N2, _NNET = 50000, 12500, 3125, 20000
_N0P, _N1P, _N2P, _NNETP = 50176, 13312, 4096, 20480

_SC_PARAMS = pltpu.CompilerParams(use_tc_tiling_on_sc=False)


def _mesh():
    return plsc.VectorSubcoreMesh(core_axis_name="c", subcore_axis_name="s")


def _pad_edges(src, dst, n_src, n_dst, ep):
    e = src.shape[0]
    extra = ep - e
    pos = jnp.arange(extra, dtype=jnp.int32)
    src_p = jnp.concatenate([src.astype(jnp.int32), pos % n_src])
    dst_p = jnp.concatenate([dst.astype(jnp.int32), n_dst + (pos % 8)])
    return src_p.reshape(ep // 128, 128), dst_p.reshape(ep // 128, 128)


def _ep(e):
    return -(-e // 16384) * 16384


def _seg_sum_sc(table2, src2d, dst2d, ndp, nsp):
    nr = src2d.shape[0]
    rpt = nr // 16
    q, t = rpt // 8, rpt % 8
    zb = ndp // 16 // 64

    def body(table, src_h, dst_h, out, srcb, dstb, rows, zblk, acc, sem):
        c = lax.axis_index("c")
        s = lax.axis_index("s")
        zv = jnp.zeros((16,), jnp.float32)
        for r in range(64):
            zblk[r, pl.ds(0, 16)] = zv
            zblk[r, pl.ds(16, 16)] = zv

        def zloop(i, carry):
            pltpu.sync_copy(zblk, acc.at[pl.ds(s * (ndp // 16) + i * 64, 64)])
            return carry

        lax.fori_loop(0, zb, zloop, 0)
        plsc.subcore_barrier()

        coff = c * nsp

        def do_batch(row0, nb):
            pltpu.sync_copy(src_h.at[pl.ds(row0, nb)], srcb.at[pl.ds(0, nb)])
            pltpu.sync_copy(dst_h.at[pl.ds(row0, nb)], dstb.at[pl.ds(0, nb)])
            for j in range(nb):
                for k in range(8):
                    srcb[j, pl.ds(k * 16, 16)] = srcb[j, pl.ds(k * 16, 16)] + coff
            cps = [pltpu.async_copy(table.at[srcb.at[j]],
                                    rows.at[pl.ds(j * 128, 128)], sem)
                   for j in range(nb)]
            for cp in cps:
                cp.wait()
            for j in range(nb):
                pltpu.sync_copy(rows.at[pl.ds(j * 128, 128)],
                                acc.at[dstb.at[j]], add=True)

        base = s * rpt

        def mloop(i, carry):
            do_batch(base + i * 8, 8)
            return carry

        lax.fori_loop(0, q, mloop, 0)
        for tt in range(t):
            do_batch(base + q * 8 + tt, 1)
        plsc.subcore_barrier()

        def oloop(i, carry):
            off = s * (ndp // 16) + i * 64
            pltpu.sync_copy(acc.at[pl.ds(off, 64)],
                            out.at[pl.ds(c * ndp + off, 64)])
            return carry

        lax.fori_loop(0, zb, oloop, 0)

    k = pl.kernel(
        body,
        out_type=jax.ShapeDtypeStruct((2 * ndp, 32), jnp.float32),
        mesh=_mesh(),
        compiler_params=_SC_PARAMS,
        scratch_types=[
            pltpu.VMEM((8, 128), jnp.int32),
            pltpu.VMEM((8, 128), jnp.int32),
            pltpu.VMEM((1024, 32), jnp.float32),
            pltpu.VMEM((64, 32), jnp.float32),
            pltpu.VMEM_SHARED((ndp, 32), jnp.float32),
            pltpu.SemaphoreType.DMA,
        ],
    )
    return k(table2, src2d, dst2d)


def _counts_sc(dsts, ctot):
    ct16 = ctot // 16
    zc = ct16 // 1024

    def body(*refs):
        dst_hs = refs[:len(dsts)]
        out, idxb, ones, zblk, acc, sem = refs[len(dsts):]
        c = lax.axis_index("c")
        s = lax.axis_index("s")
        wid = c * 16 + s
        ov = jnp.ones((16,), jnp.float32)
        zv = jnp.zeros((16,), jnp.float32)
        for k in range(8):
            ones[pl.ds(k * 16, 16)] = ov
        for k in range(64):
            zblk[pl.ds(k * 16, 16)] = zv

        def zloop(i, carry):
            pltpu.sync_copy(zblk, acc.at[pl.ds(s * ct16 + i * 1024, 1024)])
            return carry

        lax.fori_loop(0, zc, zloop, 0)
        plsc.subcore_barrier()

        for dh in dst_hs:
            rt = dh.shape[0] // 32
            qq, tt = rt // 8, rt % 8
            base = wid * rt

            def do_batch(row0, nb, dh=dh):
                pltpu.sync_copy(dh.at[pl.ds(row0, nb)], idxb.at[pl.ds(0, nb)])
                for j in range(nb):
                    pltpu.sync_copy(ones, acc.at[idxb.at[j]], add=True)

            def mloop(i, carry, base=base, do_batch=do_batch):
                do_batch(base + i * 8, 8)
                return carry

            lax.fori_loop(0, qq, mloop, 0)
            for j in range(tt):
                do_batch(base + qq * 8 + j, 1)
        plsc.subcore_barrier()

        def oloop(i, carry):
            off = s * ct16 + i * 1024
            pltpu.sync_copy(acc.at[pl.ds(off, 1024)],
                            out.at[pl.ds(c * ctot + off, 1024)])
            return carry

        lax.fori_loop(0, zc, oloop, 0)

    k = pl.kernel(
        body,
        out_type=jax.ShapeDtypeStruct((2 * ctot,), jnp.float32),
        mesh=_mesh(),
        compiler_params=_SC_PARAMS,
        scratch_types=[
            pltpu.VMEM((8, 128), jnp.int32),
            pltpu.VMEM((128,), jnp.float32),
            pltpu.VMEM((1024,), jnp.float32),
            pltpu.VMEM_SHARED((ctot,), jnp.float32),
            pltpu.SemaphoreType.DMA,
        ],
    )
    return k(*dsts)


def _seg_max_sc(table2, src2d, dst2d):
    nr = src2d.shape[0]
    nchunks = nr // 8
    rng = _NNETP // 32
    sent = rng + 7

    def body(table, src_h, dst_h, out, sstage, dstage, csrc, cdst, csrc2,
             rows0, rows1, acc0, acc1, sem):
        c = lax.axis_index("c")
        s = lax.axis_index("s")
        wid = c * 16 + s
        lo = wid * rng
        ninf = jnp.full((16,), -jnp.inf, jnp.float32)

        def init_loop(i, carry):
            acc0[i, pl.ds(0, 16)] = ninf
            acc0[i, pl.ds(16, 16)] = ninf
            acc1[i, pl.ds(0, 16)] = ninf
            acc1[i, pl.ds(16, 16)] = ninf
            return carry

        lax.fori_loop(0, rng + 8, init_loop, 0)

        def flush():
            for k in range(8):
                csrc2[pl.ds(k * 16, 16)] = csrc[pl.ds(k * 16, 16)] + _N0P
            pltpu.async_copy(table.at[csrc.at[pl.ds(0, 128)]], rows0,
                             sem).wait()
            pltpu.async_copy(table.at[csrc2.at[pl.ds(0, 128)]], rows1,
                             sem).wait()

            def upd(i, carry):
                d = cdst[i]
                acc0[d, pl.ds(0, 16)] = jnp.maximum(acc0[d, pl.ds(0, 16)],
                                                    rows0[i, pl.ds(0, 16)])
                acc0[d, pl.ds(16, 16)] = jnp.maximum(acc0[d, pl.ds(16, 16)],
                                                     rows0[i, pl.ds(16, 16)])
                acc1[d, pl.ds(0, 16)] = jnp.maximum(acc1[d, pl.ds(0, 16)],
                                                    rows1[i, pl.ds(0, 16)])
                acc1[d, pl.ds(16, 16)] = jnp.maximum(acc1[d, pl.ds(16, 16)],
                                                     rows1[i, pl.ds(16, 16)])
                return carry

            lax.fori_loop(0, 128, upd, 0)
            csrc[pl.ds(0, 16)] = csrc[pl.ds(128, 16)]
            cdst[pl.ds(0, 16)] = cdst[pl.ds(128, 16)]

        def chunk(i, cur):
            pltpu.sync_copy(src_h.at[pl.ds(i * 8, 8)], sstage)
            pltpu.sync_copy(dst_h.at[pl.ds(i * 8, 8)], dstage)
            for j in range(8):
                for k in range(8):
                    dv = dstage[j, pl.ds(k * 16, 16)]
                    sv = sstage[j, pl.ds(k * 16, 16)]
                    m = (dv >= lo) & (dv < lo + rng)
                    dl = dv - lo
                    plsc.store_compressed(cdst.at[pl.ds(cur, 16)], dl, m)
                    plsc.store_compressed(csrc.at[pl.ds(cur, 16)], sv, m)
                    cnt = jnp.max(plsc.all_reduce_population_count(m))
                    cur = cur + cnt

                    @pl.when(cur >= 128)
                    def _():
                        flush()

                    cur = jnp.where(cur >= 128, cur - 128, cur)
            return cur

        cur = lax.fori_loop(0, nchunks, chunk, jnp.int32(0))
        pos = lax.iota(jnp.int32, 16)
        for k in range(8):
            v = cdst[pl.ds(k * 16, 16)]
            cdst[pl.ds(k * 16, 16)] = jnp.where(pos + k * 16 >= cur, sent, v)
        flush()
        pltpu.sync_copy(acc0.at[pl.ds(0, rng)], out.at[pl.ds(lo, rng)])
        pltpu.sync_copy(acc1.at[pl.ds(0, rng)], out.at[pl.ds(_NNETP + lo, rng)])

    k = pl.kernel(
        body,
        out_type=jax.ShapeDtypeStruct((2 * _NNETP, 32), jnp.float32),
        mesh=_mesh(),
        compiler_params=_SC_PARAMS,
        scratch_types=[
            pltpu.VMEM((8, 128), jnp.int32),
            pltpu.VMEM((8, 128), jnp.int32),
            pltpu.VMEM((160,), jnp.int32),
            pltpu.VMEM((160,), jnp.int32),
            pltpu.VMEM((128,), jnp.int32),
            pltpu.VMEM((128, 32), jnp.float32),
            pltpu.VMEM((128, 32), jnp.float32),
            pltpu.VMEM((_NNETP // 32 + 8, 32), jnp.float32),
            pltpu.VMEM((_NNETP // 32 + 8, 32), jnp.float32),
            pltpu.SemaphoreType.DMA,
        ],
    )
    return k(table2, src2d, dst2d)


def _dotT(x, w):
    return lax.dot_general(x, w, (((1,), (1,)), ((), ())),
                           preferred_element_type=jnp.float32)


def _tc_transform(x, ws, wn, b, npad):
    bn = 512
    grid = npad // bn

    def body(x_ref, ws_ref, wn_ref, b_ref, s_ref, p_ref):
        xb = x_ref[...]
        s_ref[...] = _dotT(xb, ws_ref[...]) + b_ref[...]
        p = _dotT(xb, wn_ref[...])
        p_ref[0] = p[:, :32]
        p_ref[1] = p[:, 32:]

    return pl.pallas_call(
        body,
        grid=(grid,),
        in_specs=[
            pl.BlockSpec((bn, 64), lambda i: (i, 0)),
            pl.BlockSpec((64, 64), lambda i: (0, 0)),
            pl.BlockSpec((64, 64), lambda i: (0, 0)),
            pl.BlockSpec((1, 64), lambda i: (0, 0)),
        ],
        out_specs=[
            pl.BlockSpec((bn, 64), lambda i: (i, 0)),
            pl.BlockSpec((2, bn, 32), lambda i: (0, i, 0)),
        ],
        out_shape=[
            jax.ShapeDtypeStruct((npad, 64), jnp.float32),
            jax.ShapeDtypeStruct((2, npad, 32), jnp.float32),
        ],
    )(x, ws, wn, b.reshape(1, 64))


def _tc_combine(sarr, g, inv, npad):
    bn = 512
    grid = npad // bn

    def body(s_ref, g_ref, inv_ref, o_ref):
        gg = jnp.concatenate([g_ref[0], g_ref[1]], axis=1)
        x = jnp.tanh(s_ref[...] + gg * inv_ref[...])
        o_ref[0] = x[:, :32]
        o_ref[1] = x[:, 32:]

    return pl.pallas_call(
        body,
        grid=(grid,),
        in_specs=[
            pl.BlockSpec((bn, 64), lambda i: (i, 0)),
            pl.BlockSpec((2, bn, 32), lambda i: (0, i, 0)),
            pl.BlockSpec((bn, 1), lambda i: (i, 0)),
        ],
        out_specs=pl.BlockSpec((2, bn, 32), lambda i: (0, i, 0)),
        out_shape=jax.ShapeDtypeStruct((2, npad, 32), jnp.float32),
    )(sarr, g, inv)


def _tc_scale_transform(q, inv, ws, wn, b, npad):
    bn = 512
    grid = npad // bn

    def body(q_ref, inv_ref, ws_ref, wn_ref, b_ref, s_ref, p_ref):
        x = jnp.concatenate([q_ref[0], q_ref[1]], axis=1) * inv_ref[...]
        s_ref[...] = _dotT(x, ws_ref[...]) + b_ref[...]
        p = _dotT(x, wn_ref[...])
        p_ref[0] = p[:, :32]
        p_ref[1] = p[:, 32:]

    return pl.pallas_call(
        body,
        grid=(grid,),
        in_specs=[
            pl.BlockSpec((2, bn, 32), lambda i: (0, i, 0)),
            pl.BlockSpec((bn, 1), lambda i: (i, 0)),
            pl.BlockSpec((64, 64), lambda i: (0, 0)),
            pl.BlockSpec((64, 64), lambda i: (0, 0)),
            pl.BlockSpec((1, 64), lambda i: (0, 0)),
        ],
        out_specs=[
            pl.BlockSpec((bn, 64), lambda i: (i, 0)),
            pl.BlockSpec((2, bn, 32), lambda i: (0, i, 0)),
        ],
        out_shape=[
            jax.ShapeDtypeStruct((npad, 64), jnp.float32),
            jax.ShapeDtypeStruct((2, npad, 32), jnp.float32),
        ],
    )(q, inv, ws, wn, b.reshape(1, 64))


def _tc_up_transform(a, xt, deg, wsa, wsb, wna, wnb, b, npad):
    bn = 512
    grid = npad // bn

    def body(a_ref, x_ref, d_ref, wsa_r, wsb_r, wna_r, wnb_r, b_ref,
             s_ref, p_ref):
        aa = jnp.concatenate([a_ref[0], a_ref[1]], axis=1)
        bb = jnp.concatenate([x_ref[0], x_ref[1]], axis=1) * d_ref[...]
        s_ref[...] = _dotT(aa, wsa_r[...]) + _dotT(bb, wsb_r[...]) + b_ref[...]
        p = _dotT(aa, wna_r[...]) + _dotT(bb, wnb_r[...])
        p_ref[0] = p[:, :32]
        p_ref[1] = p[:, 32:]

    return pl.pallas_call(
        body,
        grid=(grid,),
        in_specs=[
            pl.BlockSpec((2, bn, 32), lambda i: (0, i, 0)),
            pl.BlockSpec((2, bn, 32), lambda i: (0, i, 0)),
            pl.BlockSpec((bn, 1), lambda i: (i, 0)),
            pl.BlockSpec((64, 64), lambda i: (0, 0)),
            pl.BlockSpec((64, 64), lambda i: (0, 0)),
            pl.BlockSpec((64, 64), lambda i: (0, 0)),
            pl.BlockSpec((64, 64), lambda i: (0, 0)),
            pl.BlockSpec((1, 64), lambda i: (0, 0)),
        ],
        out_specs=[
            pl.BlockSpec((bn, 64), lambda i: (i, 0)),
            pl.BlockSpec((2, bn, 32), lambda i: (0, i, 0)),
        ],
        out_shape=[
            jax.ShapeDtypeStruct((npad, 64), jnp.float32),
            jax.ShapeDtypeStruct((2, npad, 32), jnp.float32),
        ],
    )(a, xt, deg, wsa, wsb, wna, wnb, b.reshape(1, 64))


def _tc_inv(cnt2, sel, ctot):
    def body(c_ref, sel_ref, o_ref):
        tot = c_ref[0] + c_ref[1]
        o_ref[...] = jnp.where(sel_ref[...] > 0,
                               1.0 / jnp.maximum(tot, 1.0), tot)

    return pl.pallas_call(
        body,
        out_shape=jax.ShapeDtypeStruct((ctot // 128, 128), jnp.float32),
    )(cnt2.reshape(2, ctot // 128, 128), sel)


def _tc_mlp(ym, xnet, w1a, w1b, w1c, b1, w2, b2):
    bn = 512
    grid = _NNETP // bn

    def body(y_ref, xn_ref, w1a_r, w1b_r, w1c_r, b1_r, w2_r, b2_r, o_ref):
        y0 = y_ref[0]
        y1 = y_ref[1]
        y0 = jnp.where(jnp.isfinite(y0), y0, 0.0)
        y1 = jnp.where(jnp.isfinite(y1), y1, 0.0)
        h = (_dotT(y0, w1a_r[...]) + _dotT(y1, w1b_r[...])
             + _dotT(xn_ref[...], w1c_r[...]) + b1_r[...])
        h = jnp.tanh(h)
        o_ref[...] = _dotT(h, w2_r[...]) + b2_r[...]

    return pl.pallas_call(
        body,
        grid=(grid,),
        in_specs=[
            pl.BlockSpec((2, bn, 32), lambda i: (0, i, 0)),
            pl.BlockSpec((bn, 16), lambda i: (i, 0)),
            pl.BlockSpec((128, 32), lambda i: (0, 0)),
            pl.BlockSpec((128, 32), lambda i: (0, 0)),
            pl.BlockSpec((128, 16), lambda i: (0, 0)),
            pl.BlockSpec((1, 128), lambda i: (0, 0)),
            pl.BlockSpec((1, 128), lambda i: (0, 0)),
            pl.BlockSpec((1, 1), lambda i: (0, 0)),
        ],
        out_specs=pl.BlockSpec((bn, 1), lambda i: (i, 0)),
        out_shape=jax.ShapeDtypeStruct((_NNETP, 1), jnp.float32),
    )(ym, xnet, w1a, w1b, w1c, b1.reshape(1, 128), w2, b2.reshape(1, 1))


def kernel(x0, x_net, to0, to1, to2, down01_src, down01_dst, down12_src,
           down12_dst, up21_src, up21_dst, up10_src, up10_dst, conn_src,
           conn_dst, W_self_0, W_neigh_0, b_0, W_self_1, W_neigh_1, b_1,
           W_self_2, W_neigh_2, b_2, W_self_3, W_neigh_3, b_3,
           W_self_4, W_neigh_4, b_4, mlp_W1, mlp_b1, mlp_W2, mlp_b2):
    f32 = jnp.float32

    x0p = jnp.pad(x0, ((0, _N0P - _N0), (0, 0)))
    xnetp = jnp.pad(x_net, ((0, _NNETP - _NNET), (0, 0)))

    e_to0, e_to1, e_to2 = _ep(800000), _ep(200000), _ep(50000)
    e_d01, e_d12 = _ep(100000), _ep(25000)
    e_u21, e_u10, e_conn = _ep(25000), _ep(100000), _ep(400000)

    to0s, to0d = _pad_edges(to0[0], to0[1], _N0, _N0, e_to0)
    to1s, to1d = _pad_edges(to1[0], to1[1], _N1, _N1, e_to1)
    to2s, to2d = _pad_edges(to2[0], to2[1], _N2, _N2, e_to2)
    d01s, d01d = _pad_edges(down01_src, down01_dst, _N0, _N1, e_d01)
    d12s, d12d = _pad_edges(down12_src, down12_dst, _N1, _N2, e_d12)
    u21s, u21d = _pad_edges(up21_src, up21_dst, _N2, _N1, e_u21)
    u10s, u10d = _pad_edges(up10_src, up10_dst, _N1, _N0, e_u10)
    cns, cnd = _pad_edges(conn_src, conn_dst, _N0, _NNET, e_conn)

    jobs = [(to0d, 51200), (to1d, 14336), (to2d, 5120),
            (d01d, 14336), (d12d, 5120), (u21d, 14336), (u10d, 59136)]
    offs, acc_off = [], 0
    for _, ln in jobs:
        offs.append(acc_off)
        acc_off += ln
    ctot = acc_off
    cdsts = [d + o for (d, _), o in zip(jobs, offs)]
    sel = jnp.concatenate(
        [jnp.full((ln,), 1.0 if j < 5 else 0.0, f32)
         for j, (_, ln) in enumerate(jobs)]).reshape(ctot // 128, 128)

    cnt2 = _counts_sc(cdsts, ctot)
    invdeg = _tc_inv(cnt2.reshape(2, ctot), sel, ctot).reshape(ctot)

    def seg(j, npad):
        return invdeg[offs[j]:offs[j] + npad].reshape(npad, 1)

    inv_to0, inv_to1, inv_to2 = seg(0, _N0P), seg(1, _N1P), seg(2, _N2P)
    inv_d01, inv_d12 = seg(3, _N1P), seg(4, _N2P)
    deg21, deg10 = seg(5, _N1P), seg(6, _N0P)

    s0, p0 = _tc_transform(x0p, W_self_0, W_neigh_0, b_0, _N0P)
    g0 = _seg_sum_sc(p0.reshape(2 * _N0P, 32), to0s, to0d, _N0P, _N0P)
    x0t = _tc_combine(s0, g0.reshape(2, _N0P, 32), inv_to0, _N0P)

    q1 = _seg_sum_sc(x0t.reshape(2 * _N0P, 32), d01s, d01d, _N1P, _N0P)
    s1, p1 = _tc_scale_transform(q1.reshape(2, _N1P, 32), inv_d01,
                                 W_self_1, W_neigh_1, b_1, _N1P)
    g1 = _seg_sum_sc(p1.reshape(2 * _N1P, 32), to1s, to1d, _N1P, _N1P)
    x1t = _tc_combine(s1, g1.reshape(2, _N1P, 32), inv_to1, _N1P)

    q2 = _seg_sum_sc(x1t.reshape(2 * _N1P, 32), d12s, d12d, _N2P, _N1P)
    s2, p2 = _tc_scale_transform(q2.reshape(2, _N2P, 32), inv_d12,
                                 W_self_2, W_neigh_2, b_2, _N2P)
    g2 = _seg_sum_sc(p2.reshape(2 * _N2P, 32), to2s, to2d, _N2P, _N2P)
    x2t = _tc_combine(s2, g2.reshape(2, _N2P, 32), inv_to2, _N2P)

    a1 = _seg_sum_sc(x2t.reshape(2 * _N2P, 32), u21s, u21d, _N1P, _N2P)
    z1s, p3 = _tc_up_transform(a1.reshape(2, _N1P, 32), x1t, deg21,
                               W_self_3[:, :64], W_self_3[:, 64:],
                               W_neigh_3[:, :64], W_neigh_3[:, 64:],
                               b_3, _N1P)
    g3 = _seg_sum_sc(p3.reshape(2 * _N1P, 32), to1s, to1d, _N1P, _N1P)
    x1ut = _tc_combine(z1s, g3.reshape(2, _N1P, 32), inv_to1, _N1P)

    a0 = _seg_sum_sc(x1ut.reshape(2 * _N1P, 32), u10s, u10d, _N0P, _N1P)
    z0s, p4 = _tc_up_transform(a0.reshape(2, _N0P, 32), x0t, deg10,
                               W_self_4[:, :64], W_self_4[:, 64:],
                               W_neigh_4[:, :64], W_neigh_4[:, 64:],
                               b_4, _N0P)
    g4 = _seg_sum_sc(p4.reshape(2 * _N0P, 32), to0s, to0d, _N0P, _N0P)
    x0ut = _tc_combine(z0s, g4.reshape(2, _N0P, 32), inv_to0, _N0P)

    ym = _seg_max_sc(x0ut.reshape(2 * _N0P, 32), cns, cnd)
    out = _tc_mlp(ym.reshape(2, _NNETP, 32), xnetp,
                  mlp_W1[:, :32], mlp_W1[:, 32:64], mlp_W1[:, 64:],
                  mlp_b1, mlp_W2, mlp_b2)
    return out[:_NNET]


# speedup vs baseline: 3.3563x; 3.3563x over previous
import jax
import jax.numpy as jnp
from jax import lax
from jax.experimental import pallas as pl
from jax.experimental.pallas import tpu as pltpu
from jax.experimental.pallas import tpu_sc as plsc

_N0, _N1, _N2, _NNET = 50000, 12500, 3125, 20000
_N0P, _N1P, _N2P, _NNETP = 50176, 13312, 4096, 20480

_SC_PARAMS = pltpu.CompilerParams(use_tc_tiling_on_sc=False,
                                  needs_layout_passes=False)


def _mesh():
    return plsc.VectorSubcoreMesh(core_axis_name="c", subcore_axis_name="s")


def _vgather(x, idx):
    dnums = lax.GatherDimensionNumbers(offset_dims=(),
                                       collapsed_slice_dims=(0,),
                                       start_index_map=(0,))
    return lax.gather(x, idx[:, None], dnums, (1,),
                      mode=lax.GatherScatterMode.PROMISE_IN_BOUNDS)


def _pad_edges(src, dst, n_src, n_dst, ep):
    e = src.shape[0]
    extra = ep - e
    pos = jnp.arange(extra, dtype=jnp.int32)
    src_p = jnp.concatenate([src.astype(jnp.int32), pos % n_src])
    dst_p = jnp.concatenate([dst.astype(jnp.int32), n_dst + (pos % 8)])
    return src_p.reshape(ep // 128, 128), dst_p.reshape(ep // 128, 128)


def _ep(e):
    return -(-e // 32768) * 32768


def _seg_sum_sc(table2, src2d, dst2d, ndp, nsp):
    nr = src2d.shape[0]
    rpt = nr // 16
    q, t = rpt // 8, rpt % 8
    zb = ndp // 16 // 64

    def body(table, src_h, dst_h, out, srcb, dstb, rows, zblk, acc, sem):
        c = lax.axis_index("c")
        s = lax.axis_index("s")
        zv = jnp.zeros((16,), jnp.float32)
        for r in range(64):
            zblk[r, pl.ds(0, 16)] = zv

        for half in range(2):
            slab = c * 2 + half

            def zloop(i, carry):
                pltpu.sync_copy(zblk,
                                acc.at[pl.ds(s * (ndp // 16) + i * 64, 64)])
                return carry

            lax.fori_loop(0, zb, zloop, 0)
            plsc.subcore_barrier()

            coff = slab * nsp

            def do_batch(row0, nb):
                pltpu.sync_copy(src_h.at[pl.ds(row0, nb)],
                                srcb.at[pl.ds(0, nb)])
                pltpu.sync_copy(dst_h.at[pl.ds(row0, nb)],
                                dstb.at[pl.ds(0, nb)])
                for j in range(nb):
                    for k in range(8):
                        srcb[j, pl.ds(k * 16, 16)] = (
                            srcb[j, pl.ds(k * 16, 16)] + coff)
                cps = [pltpu.async_copy(table.at[srcb.at[j]],
                                        rows.at[pl.ds(j * 128, 128)], sem)
                       for j in range(nb)]
                for cp in cps:
                    cp.wait()
                for j in range(nb):
                    pltpu.sync_copy(rows.at[pl.ds(j * 128, 128)],
                                    acc.at[dstb.at[j]], add=True)

            base = s * rpt

            def mloop(i, carry):
                do_batch(base + i * 8, 8)
                return carry

            lax.fori_loop(0, q, mloop, 0)
            for tt in range(t):
                do_batch(base + q * 8 + tt, 1)
            plsc.subcore_barrier()

            def oloop(i, carry):
                off = s * (ndp // 16) + i * 64
                pltpu.sync_copy(acc.at[pl.ds(off, 64)],
                                out.at[pl.ds(slab * ndp + off, 64)])
                return carry

            lax.fori_loop(0, zb, oloop, 0)
            plsc.subcore_barrier()

    k = pl.kernel(
        body,
        out_type=jax.ShapeDtypeStruct((4 * ndp, 16), jnp.float32),
        mesh=_mesh(),
        compiler_params=_SC_PARAMS,
        scratch_types=[
            pltpu.VMEM((8, 128), jnp.int32),
            pltpu.VMEM((8, 128), jnp.int32),
            pltpu.VMEM((1024, 16), jnp.float32),
            pltpu.VMEM((64, 16), jnp.float32),
            pltpu.VMEM_SHARED((ndp, 16), jnp.float32),
            pltpu.SemaphoreType.DMA,
        ],
    )
    return k(table2, src2d, dst2d)


def _counts_sc(dsts, ctot):
    ct16 = ctot // 16
    zc = ct16 // 1024

    def body(*refs):
        dst_hs = refs[:len(dsts)]
        out, idxb, ones, zblk, acc, sem = refs[len(dsts):]
        c = lax.axis_index("c")
        s = lax.axis_index("s")
        wid = c * 16 + s
        ov = jnp.ones((16,), jnp.float32)
        zv = jnp.zeros((16,), jnp.float32)
        for k in range(8):
            ones[pl.ds(k * 16, 16)] = ov
        for k in range(64):
            zblk[pl.ds(k * 16, 16)] = zv

        def zloop(i, carry):
            off = pl.multiple_of(s * ct16 + i * 1024, 8)
            pltpu.sync_copy(zblk, acc.at[pl.ds(off, 1024)])
            return carry

        lax.fori_loop(0, zc, zloop, 0)
        plsc.subcore_barrier()

        for dh in dst_hs:
            rt = dh.shape[0] // 32
            qq, tt = rt // 8, rt % 8
            base = wid * rt

            def do_batch(row0, nb, dh=dh):
                pltpu.sync_copy(dh.at[pl.ds(row0, nb)], idxb.at[pl.ds(0, nb)])
                for j in range(nb):
                    pltpu.sync_copy(ones, acc.at[idxb.at[j]], add=True)

            def mloop(i, carry, base=base, do_batch=do_batch):
                do_batch(base + i * 8, 8)
                return carry

            lax.fori_loop(0, qq, mloop, 0)
            for j in range(tt):
                do_batch(base + qq * 8 + j, 1)
        plsc.subcore_barrier()

        def oloop(i, carry):
            off = pl.multiple_of(s * ct16 + i * 1024, 8)
            pltpu.sync_copy(acc.at[pl.ds(off, 1024)],
                            out.at[pl.ds(pl.multiple_of(
                                c * ctot + off, 8), 1024)])
            return carry

        lax.fori_loop(0, zc, oloop, 0)

    k = pl.kernel(
        body,
        out_type=jax.ShapeDtypeStruct((2 * ctot,), jnp.float32),
        mesh=_mesh(),
        compiler_params=_SC_PARAMS,
        scratch_types=[
            pltpu.VMEM((8, 128), jnp.int32),
            pltpu.VMEM((128,), jnp.float32),
            pltpu.VMEM((1024,), jnp.float32),
            pltpu.VMEM_SHARED((ctot,), jnp.float32),
            pltpu.SemaphoreType.DMA,
        ],
    )
    return k(*dsts)


def _seg_max_sc(table2, src2d, dst2d):
    nr = src2d.shape[0]
    nchunks = nr // 2 // 8
    rng = _NNETP // 16
    sent = rng

    def body(table, src_h, dst_h, out, sstage, dstage, csrc, cdst, csrc2,
             rows0, rows1, rows2, rows3, acc0, acc1, acc2, acc3, sem):
        c = lax.axis_index("c")
        s = lax.axis_index("s")
        lo = s * rng
        ninf = jnp.full((16,), -jnp.inf, jnp.float32)
        lane = lax.iota(jnp.int32, 16)
        accs = (acc0, acc1, acc2, acc3)
        rows = (rows0, rows1, rows2, rows3)

        def init_loop(i, carry):
            for a in accs:
                a[i, pl.ds(0, 16)] = ninf
            return carry

        lax.fori_loop(0, rng + 8, init_loop, 0)

        def flush_at(h0):
            h = pl.multiple_of(h0, 8)
            for q in range(4):
                if q:
                    for k in range(8):
                        csrc2[pl.ds(k * 16, 16)] = (
                            csrc[pl.ds(h + k * 16, 16)] + q * _N0P)
                    pltpu.async_copy(table.at[csrc2.at[pl.ds(0, 128)]],
                                     rows[q], sem).wait()
                else:
                    pltpu.async_copy(table.at[csrc.at[pl.ds(h, 128)]],
                                     rows[0], sem).wait()

            def upd(b, carry):
                dvec = cdst[pl.ds(h + b * 16, 16)]
                for ln in range(16):
                    d = dvec[ln]
                    i = b * 16 + ln
                    for a, r in zip(accs, rows):
                        a[d, pl.ds(0, 16)] = jnp.maximum(
                            a[d, pl.ds(0, 16)], r[i, pl.ds(0, 16)])
                return carry

            lax.fori_loop(0, 8, upd, 0)

        def append(state, sv, dv):
            cur, ncar, car_s, car_d = state
            m = (dv >= lo) & (dv < lo + rng)
            dl = dv - lo
            keys = lane + jnp.where(m, 0, 16)
            _, perm = plsc.sort_key_val(keys, lane)
            nv = jnp.max(plsc.all_reduce_population_count(m))
            csv = _vgather(sv, perm)
            cdl = _vgather(dl, perm)
            i1 = jnp.clip(lane - ncar, 0, 15)
            f_s = jnp.where(lane < ncar, car_s,
                            _vgather(csv, i1))
            f_d = jnp.where(lane < ncar, car_d,
                            _vgather(cdl, i1))
            total = ncar + nv
            full = total >= 16

            @pl.when(full)
            def _():
                cc = pl.multiple_of(cur, 8)
                csrc[pl.ds(cc, 16)] = f_s
                cdst[pl.ds(cc, 16)] = f_d

            i2 = jnp.clip(lane + 16 - ncar, 0, 15)
            g_s = _vgather(csv, i2)
            g_d = _vgather(cdl, i2)
            car_s = jnp.where(full, g_s, f_s)
            car_d = jnp.where(full, g_d, f_d)
            step = jnp.where(full, 16, 0)
            return cur + step, total - step, car_s, car_d

        def chunk(i, state):
            base = c * (nr // 2) + i * 8
            pltpu.sync_copy(src_h.at[pl.ds(base, 8)], sstage)
            pltpu.sync_copy(dst_h.at[pl.ds(base, 8)], dstage)
            for j in range(8):
                for k in range(8):
                    dv = dstage[j, pl.ds(k * 16, 16)]
                    sv = sstage[j, pl.ds(k * 16, 16)]
                    state = append(state, sv, dv)
            cur, ncar, car_s, car_d = state

            def wbody(h):
                flush_at(h)
                return h + 128

            h = lax.while_loop(lambda h: h + 128 <= cur, wbody, jnp.int32(0))
            hh = pl.multiple_of(h, 8)
            for k in range(8):
                csrc[pl.ds(k * 16, 16)] = csrc[pl.ds(hh + k * 16, 16)]
                cdst[pl.ds(k * 16, 16)] = cdst[pl.ds(hh + k * 16, 16)]
            return cur - h, ncar, car_s, car_d

        state0 = (jnp.int32(0), jnp.int32(0),
                  jnp.zeros((16,), jnp.int32), jnp.full((16,), sent, jnp.int32))
        cur, ncar, car_s, car_d = lax.fori_loop(0, nchunks, chunk, state0)
        cc = pl.multiple_of(cur, 8)
        csrc[pl.ds(cc, 16)] = car_s
        cdst[pl.ds(cc, 16)] = jnp.where(lane < ncar, car_d, sent)
        nedges = cur + ncar
        for k in range(8):
            v = cdst[pl.ds(k * 16, 16)]
            w = csrc[pl.ds(k * 16, 16)]
            tail = lane + k * 16 >= nedges
            cdst[pl.ds(k * 16, 16)] = jnp.where(tail, sent, v)
            csrc[pl.ds(k * 16, 16)] = jnp.where(tail, 0, w)
        flush_at(jnp.int32(0))
        for q in range(4):
            pltpu.sync_copy(
                accs[q].at[pl.ds(0, rng)],
                out.at[pl.ds((c * 4 + q) * _NNETP + lo, rng)])

    k = pl.kernel(
        body,
        out_type=jax.ShapeDtypeStruct((8 * _NNETP, 16), jnp.float32),
        mesh=_mesh(),
        compiler_params=_SC_PARAMS,
        scratch_types=[
            pltpu.VMEM((8, 128), jnp.int32),
            pltpu.VMEM((8, 128), jnp.int32),
            pltpu.VMEM((1184,), jnp.int32),
            pltpu.VMEM((1184,), jnp.int32),
            pltpu.VMEM((128,), jnp.int32),
            pltpu.VMEM((128, 16), jnp.float32),
            pltpu.VMEM((128, 16), jnp.float32),
            pltpu.VMEM((128, 16), jnp.float32),
            pltpu.VMEM((128, 16), jnp.float32),
            pltpu.VMEM((_NNETP // 16 + 8, 16), jnp.float32),
            pltpu.VMEM((_NNETP // 16 + 8, 16), jnp.float32),
            pltpu.VMEM((_NNETP // 16 + 8, 16), jnp.float32),
            pltpu.VMEM((_NNETP // 16 + 8, 16), jnp.float32),
            pltpu.SemaphoreType.DMA,
        ],
    )
    return k(table2, src2d, dst2d)


def _dotT(x, w):
    return lax.dot_general(x, w, (((1,), (1,)), ((), ())),
                           preferred_element_type=jnp.float32)


def _split_store(p_ref, p):
    for qq in range(4):
        p_ref[qq] = p[:, 16 * qq:16 * (qq + 1)]


def _cat(ref):
    return jnp.concatenate([ref[0], ref[1], ref[2], ref[3]], axis=1)


def _wspec():
    return pl.BlockSpec((64, 64), lambda i: (0, 0))


def _tspec(bn):
    return pl.BlockSpec((4, bn, 16), lambda i: (0, i, 0))


def _tc_transform(x, ws, wn, b, npad):
    bn = 512

    def body(x_ref, ws_ref, wn_ref, b_ref, s_ref, p_ref):
        xb = x_ref[...]
        s_ref[...] = _dotT(xb, ws_ref[...]) + b_ref[...]
        _split_store(p_ref, _dotT(xb, wn_ref[...]))

    return pl.pallas_call(
        body,
        grid=(npad // bn,),
        in_specs=[
            pl.BlockSpec((bn, 64), lambda i: (i, 0)),
            _wspec(), _wspec(),
            pl.BlockSpec((1, 64), lambda i: (0, 0)),
        ],
        out_specs=[pl.BlockSpec((bn, 64), lambda i: (i, 0)), _tspec(bn)],
        out_shape=[
            jax.ShapeDtypeStruct((npad, 64), jnp.float32),
            jax.ShapeDtypeStruct((4, npad, 16), jnp.float32),
        ],
    )(x, ws, wn, b.reshape(1, 64))


def _tc_combine(sarr, g, inv, npad):
    bn = 512

    def body(s_ref, g_ref, inv_ref, o_ref):
        x = jnp.tanh(s_ref[...] + _cat(g_ref) * inv_ref[...])
        _split_store(o_ref, x)

    return pl.pallas_call(
        body,
        grid=(npad // bn,),
        in_specs=[
            pl.BlockSpec((bn, 64), lambda i: (i, 0)),
            _tspec(bn),
            pl.BlockSpec((bn, 1), lambda i: (i, 0)),
        ],
        out_specs=_tspec(bn),
        out_shape=jax.ShapeDtypeStruct((4, npad, 16), jnp.float32),
    )(sarr, g, inv)


def _tc_scale_transform(q, inv, ws, wn, b, npad):
    bn = 512

    def body(q_ref, inv_ref, ws_ref, wn_ref, b_ref, s_ref, p_ref):
        x = _cat(q_ref) * inv_ref[...]
        s_ref[...] = _dotT(x, ws_ref[...]) + b_ref[...]
        _split_store(p_ref, _dotT(x, wn_ref[...]))

    return pl.pallas_call(
        body,
        grid=(npad // bn,),
        in_specs=[
            _tspec(bn),
            pl.BlockSpec((bn, 1), lambda i: (i, 0)),
            _wspec(), _wspec(),
            pl.BlockSpec((1, 64), lambda i: (0, 0)),
        ],
        out_specs=[pl.BlockSpec((bn, 64), lambda i: (i, 0)), _tspec(bn)],
        out_shape=[
            jax.ShapeDtypeStruct((npad, 64), jnp.float32),
            jax.ShapeDtypeStruct((4, npad, 16), jnp.float32),
        ],
    )(q, inv, ws, wn, b.reshape(1, 64))


def _tc_up_transform(a, xt, deg, wsa, wsb, wna, wnb, b, npad):
    bn = 512

    def body(a_ref, x_ref, d_ref, wsa_r, wsb_r, wna_r, wnb_r, b_ref,
             s_ref, p_ref):
        aa = _cat(a_ref)
        bb = _cat(x_ref) * d_ref[...]
        s_ref[...] = (_dotT(aa, wsa_r[...]) + _dotT(bb, wsb_r[...])
                      + b_ref[...])
        _split_store(p_ref, _dotT(aa, wna_r[...]) + _dotT(bb, wnb_r[...]))

    return pl.pallas_call(
        body,
        grid=(npad // bn,),
        in_specs=[
            _tspec(bn), _tspec(bn),
            pl.BlockSpec((bn, 1), lambda i: (i, 0)),
            _wspec(), _wspec(), _wspec(), _wspec(),
            pl.BlockSpec((1, 64), lambda i: (0, 0)),
        ],
        out_specs=[pl.BlockSpec((bn, 64), lambda i: (i, 0)), _tspec(bn)],
        out_shape=[
            jax.ShapeDtypeStruct((npad, 64), jnp.float32),
            jax.ShapeDtypeStruct((4, npad, 16), jnp.float32),
        ],
    )(a, xt, deg, wsa, wsb, wna, wnb, b.reshape(1, 64))


def _tc_inv(cnt2, sel, ctot):
    def body(c_ref, sel_ref, o_ref):
        tot = c_ref[0] + c_ref[1]
        o_ref[...] = jnp.where(sel_ref[...] > 0,
                               1.0 / jnp.maximum(tot, 1.0), tot)

    return pl.pallas_call(
        body,
        out_shape=jax.ShapeDtypeStruct((ctot // 128, 128), jnp.float32),
    )(cnt2.reshape(2, ctot // 128, 128), sel)


def _tc_mlp(ym, xnet, w1a, w1b, w1c, b1, w2, b2):
    bn = 512

    def body(y_ref, xn_ref, w1a_r, w1b_r, w1c_r, b1_r, w2_r, b2_r, o_ref):
        ya = jnp.maximum(y_ref[0], y_ref[1])
        y0 = jnp.concatenate([ya[0], ya[1]], axis=1)
        y1 = jnp.concatenate([ya[2], ya[3]], axis=1)
        y0 = jnp.where(jnp.isfinite(y0), y0, 0.0)
        y1 = jnp.where(jnp.isfinite(y1), y1, 0.0)
        h = (_dotT(y0, w1a_r[...]) + _dotT(y1, w1b_r[...])
             + _dotT(xn_ref[...], w1c_r[...]) + b1_r[...])
        h = jnp.tanh(h)
        o_ref[...] = (jnp.sum(h * w2_r[...], axis=1, keepdims=True)
                      + b2_r[0, 0])

    return pl.pallas_call(
        body,
        grid=(_NNETP // bn,),
        in_specs=[
            pl.BlockSpec((2, 4, bn, 16), lambda i: (0, 0, i, 0)),
            pl.BlockSpec((bn, 16), lambda i: (i, 0)),
            pl.BlockSpec((128, 32), lambda i: (0, 0)),
            pl.BlockSpec((128, 32), lambda i: (0, 0)),
            pl.BlockSpec((128, 16), lambda i: (0, 0)),
            pl.BlockSpec((1, 128), lambda i: (0, 0)),
            pl.BlockSpec((1, 128), lambda i: (0, 0)),
            pl.BlockSpec((1, 1), lambda i: (0, 0)),
        ],
        out_specs=pl.BlockSpec((bn, 1), lambda i: (i, 0)),
        out_shape=jax.ShapeDtypeStruct((_NNETP, 1), jnp.float32),
    )(ym, xnet, w1a, w1b, w1c, b1.reshape(1, 128), w2, b2.reshape(1, 1))


def kernel(x0, x_net, to0, to1, to2, down01_src, down01_dst, down12_src,
           down12_dst, up21_src, up21_dst, up10_src, up10_dst, conn_src,
           conn_dst, W_self_0, W_neigh_0, b_0, W_self_1, W_neigh_1, b_1,
           W_self_2, W_neigh_2, b_2, W_self_3, W_neigh_3, b_3,
           W_self_4, W_neigh_4, b_4, mlp_W1, mlp_b1, mlp_W2, mlp_b2):
    f32 = jnp.float32

    x0p = jnp.pad(x0, ((0, _N0P - _N0), (0, 0)))
    xnetp = jnp.pad(x_net, ((0, _NNETP - _NNET), (0, 0)))

    to0s, to0d = _pad_edges(to0[0], to0[1], _N0, _N0, _ep(800000))
    to1s, to1d = _pad_edges(to1[0], to1[1], _N1, _N1, _ep(200000))
    to2s, to2d = _pad_edges(to2[0], to2[1], _N2, _N2, _ep(50000))
    d01s, d01d = _pad_edges(down01_src, down01_dst, _N0, _N1, _ep(100000))
    d12s, d12d = _pad_edges(down12_src, down12_dst, _N1, _N2, _ep(25000))
    u21s, u21d = _pad_edges(up21_src, up21_dst, _N2, _N1, _ep(25000))
    u10s, u10d = _pad_edges(up10_src, up10_dst, _N1, _N0, _ep(100000))
    cns, cnd = _pad_edges(conn_src, conn_dst, _N0, _NNET, _ep(400000))

    jobs = [(to0d, 51200), (to1d, 14336), (to2d, 5120),
            (d01d, 14336), (d12d, 5120), (u21d, 14336), (u10d, 59392)]
    offs, acc_off = [], 0
    for _, ln in jobs:
        offs.append(acc_off)
        acc_off += ln
    ctot = acc_off
    cdsts = [d + o for (d, _), o in zip(jobs, offs)]
    sel = jnp.concatenate(
        [jnp.full((ln,), 1.0 if j < 5 else 0.0, f32)
         for j, (_, ln) in enumerate(jobs)]).reshape(ctot // 128, 128)

    _DEBUG_JNP_COUNTS = False
    if _DEBUG_JNP_COUNTS:
        def _cnt(d, n, npad):
            c = jax.ops.segment_sum(jnp.ones(d.shape, f32), d, num_segments=n)
            return jnp.pad(c, (0, npad - n)).reshape(npad, 1)
        inv_to0 = 1.0 / jnp.maximum(_cnt(to0[1], _N0, _N0P), 1.0)
        inv_to1 = 1.0 / jnp.maximum(_cnt(to1[1], _N1, _N1P), 1.0)
        inv_to2 = 1.0 / jnp.maximum(_cnt(to2[1], _N2, _N2P), 1.0)
        inv_d01 = 1.0 / jnp.maximum(_cnt(down01_dst, _N1, _N1P), 1.0)
        inv_d12 = 1.0 / jnp.maximum(_cnt(down12_dst, _N2, _N2P), 1.0)
        deg21 = _cnt(up21_dst, _N1, _N1P)
        deg10 = _cnt(up10_dst, _N0, _N0P)
    else:
        cnt2 = _counts_sc(cdsts, ctot)
        invdeg = _tc_inv(cnt2.reshape(2, ctot), sel, ctot).reshape(ctot)

        def seg(j, npad):
            return invdeg[offs[j]:offs[j] + npad].reshape(npad, 1)

        inv_to0, inv_to1, inv_to2 = seg(0, _N0P), seg(1, _N1P), seg(2, _N2P)
        inv_d01, inv_d12 = seg(3, _N1P), seg(4, _N2P)
        deg21, deg10 = seg(5, _N1P), seg(6, _N0P)

    def flat(p):
        return p.reshape(p.shape[0] * p.shape[1], p.shape[2])

    s0, p0 = _tc_transform(x0p, W_self_0, W_neigh_0, b_0, _N0P)
    g0 = _seg_sum_sc(flat(p0), to0s, to0d, _N0P, _N0P)
    x0t = _tc_combine(s0, g0.reshape(4, _N0P, 16), inv_to0, _N0P)

    q1 = _seg_sum_sc(flat(x0t), d01s, d01d, _N1P, _N0P)
    s1, p1 = _tc_scale_transform(q1.reshape(4, _N1P, 16), inv_d01,
                                 W_self_1, W_neigh_1, b_1, _N1P)
    g1 = _seg_sum_sc(flat(p1), to1s, to1d, _N1P, _N1P)
    x1t = _tc_combine(s1, g1.reshape(4, _N1P, 16), inv_to1, _N1P)

    q2 = _seg_sum_sc(flat(x1t), d12s, d12d, _N2P, _N1P)
    s2, p2 = _tc_scale_transform(q2.reshape(4, _N2P, 16), inv_d12,
                                 W_self_2, W_neigh_2, b_2, _N2P)
    g2 = _seg_sum_sc(flat(p2), to2s, to2d, _N2P, _N2P)
    x2t = _tc_combine(s2, g2.reshape(4, _N2P, 16), inv_to2, _N2P)

    a1 = _seg_sum_sc(flat(x2t), u21s, u21d, _N1P, _N2P)
    z1s, p3 = _tc_up_transform(a1.reshape(4, _N1P, 16), x1t, deg21,
                               W_self_3[:, :64], W_self_3[:, 64:],
                               W_neigh_3[:, :64], W_neigh_3[:, 64:],
                               b_3, _N1P)
    g3 = _seg_sum_sc(flat(p3), to1s, to1d, _N1P, _N1P)
    x1ut = _tc_combine(z1s, g3.reshape(4, _N1P, 16), inv_to1, _N1P)

    a0 = _seg_sum_sc(flat(x1ut), u10s, u10d, _N0P, _N1P)
    z0s, p4 = _tc_up_transform(a0.reshape(4, _N0P, 16), x0t, deg10,
                               W_self_4[:, :64], W_self_4[:, 64:],
                               W_neigh_4[:, :64], W_neigh_4[:, 64:],
                               b_4, _N0P)
    g4 = _seg_sum_sc(flat(p4), to0s, to0d, _N0P, _N0P)
    x0ut = _tc_combine(z0s, g4.reshape(4, _N0P, 16), inv_to0, _N0P)

    _DEBUG_JNP_READOUT = False
    if _DEBUG_JNP_READOUT:
        xu = jnp.concatenate([x0ut[q] for q in range(4)], axis=1)[:_N0]
        x1p, x2p = xu[:, :32], xu[:, 32:]
        y_max = jax.ops.segment_max(jnp.take(x1p, conn_src, axis=0),
                                    conn_dst, num_segments=_NNET)
        y_max = jnp.where(jnp.isfinite(y_max), y_max, 0.0)
        y_min = jax.ops.segment_max(jnp.take(x2p, conn_src, axis=0),
                                    conn_dst, num_segments=_NNET)
        y_min = jnp.where(jnp.isfinite(y_min), y_min, 0.0)
        xx = jnp.concatenate([y_max, y_min, x_net], axis=1)
        xx = jnp.tanh(xx @ mlp_W1.T + mlp_b1)
        return xx @ mlp_W2.T + mlp_b2
    ym = _seg_max_sc(flat(x0ut), cns, cnd)
    out = _tc_mlp(ym.reshape(2, 4, _NNETP, 16), xnetp,
                  mlp_W1[:, :32], mlp_W1[:, 32:64], mlp_W1[:, 64:],
                  mlp_b1, mlp_W2, mlp_b2)
    return out[:_NNET]


# speedup vs baseline: 3.5291x; 1.0515x over previous
import jax
import jax.numpy as jnp
from jax import lax
from jax.experimental import pallas as pl
from jax.experimental.pallas import tpu as pltpu
from jax.experimental.pallas import tpu_sc as plsc

_N0, _N1, _N2, _NNET = 50000, 12500, 3125, 20000
_N0P, _N1P, _N2P, _NNETP = 50176, 13312, 4096, 20480

_SC_PARAMS = pltpu.CompilerParams(use_tc_tiling_on_sc=False,
                                  needs_layout_passes=False)


def _mesh():
    return plsc.VectorSubcoreMesh(core_axis_name="c", subcore_axis_name="s")


def _vgather(x, idx):
    dnums = lax.GatherDimensionNumbers(offset_dims=(),
                                       collapsed_slice_dims=(0,),
                                       start_index_map=(0,))
    return lax.gather(x, idx[:, None], dnums, (1,),
                      mode=lax.GatherScatterMode.PROMISE_IN_BOUNDS)


def _pad_edges(src, dst, n_src, n_dst, ep):
    e = src.shape[0]
    extra = ep - e
    pos = jnp.arange(extra, dtype=jnp.int32)
    src_p = jnp.concatenate([src.astype(jnp.int32), pos % n_src])
    dst_p = jnp.concatenate([dst.astype(jnp.int32), n_dst + (pos % 8)])
    return src_p.reshape(ep // 128, 128), dst_p.reshape(ep // 128, 128)


def _ep(e):
    return -(-e // 32768) * 32768


def _seg_sum_sc(table2, src2d, dst2d, ndp, nsp):
    nr = src2d.shape[0]
    rpt = nr // 16
    ki = 16
    q, t = rpt // ki, rpt % ki
    zb = ndp // 16 // 64

    def body(table, src_h, dst_h, out, srcb, dstb, rows, zblk, acc, sem):
        c = lax.axis_index("c")
        s = lax.axis_index("s")
        zv = jnp.zeros((16,), jnp.float32)
        for r in range(64):
            zblk[r, pl.ds(0, 16)] = zv

        for half in range(2):
            slab = c * 2 + half

            def zloop(i, carry):
                pltpu.sync_copy(zblk,
                                acc.at[pl.ds(s * (ndp // 16) + i * 64, 64)])
                return carry

            lax.fori_loop(0, zb, zloop, 0)
            plsc.subcore_barrier()

            coff = slab * nsp

            def do_batch(row0, nb):
                pltpu.sync_copy(src_h.at[pl.ds(row0, nb)],
                                srcb.at[pl.ds(0, nb)])
                pltpu.sync_copy(dst_h.at[pl.ds(row0, nb)],
                                dstb.at[pl.ds(0, nb)])
                for j in range(nb):
                    for k in range(8):
                        srcb[j, pl.ds(k * 16, 16)] = (
                            srcb[j, pl.ds(k * 16, 16)] + coff)
                cps = [pltpu.async_copy(table.at[srcb.at[j]],
                                        rows.at[pl.ds(j * 128, 128)], sem)
                       for j in range(nb)]
                for cp in cps:
                    cp.wait()
                for j in range(nb):
                    pltpu.sync_copy(rows.at[pl.ds(j * 128, 128)],
                                    acc.at[dstb.at[j]], add=True)

            base = s * rpt

            def mloop(i, carry):
                do_batch(base + i * ki, ki)
                return carry

            lax.fori_loop(0, q, mloop, 0)
            for tt in range(t):
                do_batch(base + q * ki + tt, 1)
            plsc.subcore_barrier()

            def oloop(i, carry):
                off = s * (ndp // 16) + i * 64
                pltpu.sync_copy(acc.at[pl.ds(off, 64)],
                                out.at[pl.ds(slab * ndp + off, 64)])
                return carry

            lax.fori_loop(0, zb, oloop, 0)
            plsc.subcore_barrier()

    k = pl.kernel(
        body,
        out_type=jax.ShapeDtypeStruct((4 * ndp, 16), jnp.float32),
        mesh=_mesh(),
        compiler_params=_SC_PARAMS,
        scratch_types=[
            pltpu.VMEM((16, 128), jnp.int32),
            pltpu.VMEM((16, 128), jnp.int32),
            pltpu.VMEM((2048, 16), jnp.float32),
            pltpu.VMEM((64, 16), jnp.float32),
            pltpu.VMEM_SHARED((ndp, 16), jnp.float32),
            pltpu.SemaphoreType.DMA,
        ],
    )
    return k(table2, src2d, dst2d)


def _counts_sc(dsts, ctot):
    ct16 = ctot // 16
    zc = ct16 // 1024

    def body(*refs):
        dst_hs = refs[:len(dsts)]
        out, idxb, ones, zblk, acc, sem = refs[len(dsts):]
        c = lax.axis_index("c")
        s = lax.axis_index("s")
        wid = c * 16 + s
        ov = jnp.ones((16,), jnp.float32)
        zv = jnp.zeros((16,), jnp.float32)
        for k in range(8):
            ones[pl.ds(k * 16, 16)] = ov
        for k in range(64):
            zblk[pl.ds(k * 16, 16)] = zv

        def zloop(i, carry):
            off = pl.multiple_of(s * ct16 + i * 1024, 8)
            pltpu.sync_copy(zblk, acc.at[pl.ds(off, 1024)])
            return carry

        lax.fori_loop(0, zc, zloop, 0)
        plsc.subcore_barrier()

        for dh in dst_hs:
            rt = dh.shape[0] // 32
            qq, tt = rt // 8, rt % 8
            base = wid * rt

            def do_batch(row0, nb, dh=dh):
                pltpu.sync_copy(dh.at[pl.ds(row0, nb)], idxb.at[pl.ds(0, nb)])
                for j in range(nb):
                    pltpu.sync_copy(ones, acc.at[idxb.at[j]], add=True)

            def mloop(i, carry, base=base, do_batch=do_batch):
                do_batch(base + i * 8, 8)
                return carry

            lax.fori_loop(0, qq, mloop, 0)
            for j in range(tt):
                do_batch(base + qq * 8 + j, 1)
        plsc.subcore_barrier()

        def oloop(i, carry):
            off = pl.multiple_of(s * ct16 + i * 1024, 8)
            pltpu.sync_copy(acc.at[pl.ds(off, 1024)],
                            out.at[pl.ds(pl.multiple_of(
                                c * ctot + off, 8), 1024)])
            return carry

        lax.fori_loop(0, zc, oloop, 0)

    k = pl.kernel(
        body,
        out_type=jax.ShapeDtypeStruct((2 * ctot,), jnp.float32),
        mesh=_mesh(),
        compiler_params=_SC_PARAMS,
        scratch_types=[
            pltpu.VMEM((8, 128), jnp.int32),
            pltpu.VMEM((128,), jnp.float32),
            pltpu.VMEM((1024,), jnp.float32),
            pltpu.VMEM_SHARED((ctot,), jnp.float32),
            pltpu.SemaphoreType.DMA,
        ],
    )
    return k(*dsts)


def _seg_max_sc(table2, src2d, dst2d):
    nr = src2d.shape[0]
    nchunks = nr // 2 // 8
    rng = _NNETP // 16
    sent = rng

    def body(table, src_h, dst_h, out, sstage, dstage, csrc, cdst, csrc2,
             rows0, rows1, rows2, rows3, acc0, acc1, acc2, acc3, sem):
        c = lax.axis_index("c")
        s = lax.axis_index("s")
        lo = s * rng
        ninf = jnp.full((16,), -jnp.inf, jnp.float32)
        lane = lax.iota(jnp.int32, 16)
        accs = (acc0, acc1, acc2, acc3)
        rows = (rows0, rows1, rows2, rows3)

        def init_loop(i, carry):
            for a in accs:
                a[i, pl.ds(0, 16)] = ninf
            return carry

        lax.fori_loop(0, rng + 8, init_loop, 0)

        def flush_at(h0):
            h = pl.multiple_of(h0, 8)
            for q in range(4):
                if q:
                    for k in range(8):
                        csrc2[pl.ds(k * 16, 16)] = (
                            csrc[pl.ds(h + k * 16, 16)] + q * _N0P)
                    pltpu.async_copy(table.at[csrc2.at[pl.ds(0, 128)]],
                                     rows[q], sem).wait()
                else:
                    pltpu.async_copy(table.at[csrc.at[pl.ds(h, 128)]],
                                     rows[0], sem).wait()

            def upd(b, carry):
                dvec = cdst[pl.ds(h + b * 16, 16)]
                for ln in range(16):
                    d = dvec[ln]
                    i = b * 16 + ln
                    for a, r in zip(accs, rows):
                        a[d, pl.ds(0, 16)] = jnp.maximum(
                            a[d, pl.ds(0, 16)], r[i, pl.ds(0, 16)])
                return carry

            lax.fori_loop(0, 8, upd, 0)

        def append(state, sv, dv):
            cur, ncar, car_s, car_d = state
            m = (dv >= lo) & (dv < lo + rng)
            dl = dv - lo
            keys = lane + jnp.where(m, 0, 16)
            _, perm = plsc.sort_key_val(keys, lane)
            nv = jnp.max(plsc.all_reduce_population_count(m))
            csv = _vgather(sv, perm)
            cdl = _vgather(dl, perm)
            i1 = jnp.clip(lane - ncar, 0, 15)
            f_s = jnp.where(lane < ncar, car_s,
                            _vgather(csv, i1))
            f_d = jnp.where(lane < ncar, car_d,
                            _vgather(cdl, i1))
            total = ncar + nv
            full = total >= 16

            @pl.when(full)
            def _():
                cc = pl.multiple_of(cur, 8)
                csrc[pl.ds(cc, 16)] = f_s
                cdst[pl.ds(cc, 16)] = f_d

            i2 = jnp.clip(lane + 16 - ncar, 0, 15)
            g_s = _vgather(csv, i2)
            g_d = _vgather(cdl, i2)
            car_s = jnp.where(full, g_s, f_s)
            car_d = jnp.where(full, g_d, f_d)
            step = jnp.where(full, 16, 0)
            return cur + step, total - step, car_s, car_d

        def chunk(i, state):
            base = c * (nr // 2) + i * 8
            pltpu.sync_copy(src_h.at[pl.ds(base, 8)], sstage)
            pltpu.sync_copy(dst_h.at[pl.ds(base, 8)], dstage)
            for j in range(8):
                for k in range(8):
                    dv = dstage[j, pl.ds(k * 16, 16)]
                    sv = sstage[j, pl.ds(k * 16, 16)]
                    state = append(state, sv, dv)
            cur, ncar, car_s, car_d = state

            def wbody(h):
                flush_at(h)
                return h + 128

            h = lax.while_loop(lambda h: h + 128 <= cur, wbody, jnp.int32(0))
            hh = pl.multiple_of(h, 8)
            for k in range(8):
                csrc[pl.ds(k * 16, 16)] = csrc[pl.ds(hh + k * 16, 16)]
                cdst[pl.ds(k * 16, 16)] = cdst[pl.ds(hh + k * 16, 16)]
            return cur - h, ncar, car_s, car_d

        state0 = (jnp.int32(0), jnp.int32(0),
                  jnp.zeros((16,), jnp.int32), jnp.full((16,), sent, jnp.int32))
        cur, ncar, car_s, car_d = lax.fori_loop(0, nchunks, chunk, state0)
        cc = pl.multiple_of(cur, 8)
        csrc[pl.ds(cc, 16)] = car_s
        cdst[pl.ds(cc, 16)] = jnp.where(lane < ncar, car_d, sent)
        nedges = cur + ncar
        for k in range(8):
            v = cdst[pl.ds(k * 16, 16)]
            w = csrc[pl.ds(k * 16, 16)]
            tail = lane + k * 16 >= nedges
            cdst[pl.ds(k * 16, 16)] = jnp.where(tail, sent, v)
            csrc[pl.ds(k * 16, 16)] = jnp.where(tail, 0, w)
        flush_at(jnp.int32(0))
        for q in range(4):
            pltpu.sync_copy(
                accs[q].at[pl.ds(0, rng)],
                out.at[pl.ds((c * 4 + q) * _NNETP + lo, rng)])

    k = pl.kernel(
        body,
        out_type=jax.ShapeDtypeStruct((8 * _NNETP, 16), jnp.float32),
        mesh=_mesh(),
        compiler_params=_SC_PARAMS,
        scratch_types=[
            pltpu.VMEM((8, 128), jnp.int32),
            pltpu.VMEM((8, 128), jnp.int32),
            pltpu.VMEM((1184,), jnp.int32),
            pltpu.VMEM((1184,), jnp.int32),
            pltpu.VMEM((128,), jnp.int32),
            pltpu.VMEM((128, 16), jnp.float32),
            pltpu.VMEM((128, 16), jnp.float32),
            pltpu.VMEM((128, 16), jnp.float32),
            pltpu.VMEM((128, 16), jnp.float32),
            pltpu.VMEM((_NNETP // 16 + 8, 16), jnp.float32),
            pltpu.VMEM((_NNETP // 16 + 8, 16), jnp.float32),
            pltpu.VMEM((_NNETP // 16 + 8, 16), jnp.float32),
            pltpu.VMEM((_NNETP // 16 + 8, 16), jnp.float32),
            pltpu.SemaphoreType.DMA,
        ],
    )
    return k(table2, src2d, dst2d)


def _dotT(x, w):
    return lax.dot_general(x, w, (((1,), (1,)), ((), ())),
                           preferred_element_type=jnp.float32)


def _split_store(p_ref, p):
    for qq in range(4):
        p_ref[qq] = p[:, 16 * qq:16 * (qq + 1)]


def _cat(ref):
    return jnp.concatenate([ref[0], ref[1], ref[2], ref[3]], axis=1)


def _wspec():
    return pl.BlockSpec((64, 64), lambda i: (0, 0))


def _tspec(bn):
    return pl.BlockSpec((4, bn, 16), lambda i: (0, i, 0))


def _tc_transform(x, ws, wn, b, npad):
    bn = 512

    def body(x_ref, ws_ref, wn_ref, b_ref, s_ref, p_ref):
        xb = x_ref[...]
        s_ref[...] = _dotT(xb, ws_ref[...]) + b_ref[...]
        _split_store(p_ref, _dotT(xb, wn_ref[...]))

    return pl.pallas_call(
        body,
        grid=(npad // bn,),
        in_specs=[
            pl.BlockSpec((bn, 64), lambda i: (i, 0)),
            _wspec(), _wspec(),
            pl.BlockSpec((1, 64), lambda i: (0, 0)),
        ],
        out_specs=[pl.BlockSpec((bn, 64), lambda i: (i, 0)), _tspec(bn)],
        out_shape=[
            jax.ShapeDtypeStruct((npad, 64), jnp.float32),
            jax.ShapeDtypeStruct((4, npad, 16), jnp.float32),
        ],
    )(x, ws, wn, b.reshape(1, 64))


def _tc_combine(sarr, g, inv, npad):
    bn = 512

    def body(s_ref, g_ref, inv_ref, o_ref):
        x = jnp.tanh(s_ref[...] + _cat(g_ref) * inv_ref[...])
        _split_store(o_ref, x)

    return pl.pallas_call(
        body,
        grid=(npad // bn,),
        in_specs=[
            pl.BlockSpec((bn, 64), lambda i: (i, 0)),
            _tspec(bn),
            pl.BlockSpec((bn, 1), lambda i: (i, 0)),
        ],
        out_specs=_tspec(bn),
        out_shape=jax.ShapeDtypeStruct((4, npad, 16), jnp.float32),
    )(sarr, g, inv)


def _tc_scale_transform(q, inv, ws, wn, b, npad):
    bn = 512

    def body(q_ref, inv_ref, ws_ref, wn_ref, b_ref, s_ref, p_ref):
        x = _cat(q_ref) * inv_ref[...]
        s_ref[...] = _dotT(x, ws_ref[...]) + b_ref[...]
        _split_store(p_ref, _dotT(x, wn_ref[...]))

    return pl.pallas_call(
        body,
        grid=(npad // bn,),
        in_specs=[
            _tspec(bn),
            pl.BlockSpec((bn, 1), lambda i: (i, 0)),
            _wspec(), _wspec(),
            pl.BlockSpec((1, 64), lambda i: (0, 0)),
        ],
        out_specs=[pl.BlockSpec((bn, 64), lambda i: (i, 0)), _tspec(bn)],
        out_shape=[
            jax.ShapeDtypeStruct((npad, 64), jnp.float32),
            jax.ShapeDtypeStruct((4, npad, 16), jnp.float32),
        ],
    )(q, inv, ws, wn, b.reshape(1, 64))


def _tc_up_transform(a, xt, deg, wsa, wsb, wna, wnb, b, npad):
    bn = 512

    def body(a_ref, x_ref, d_ref, wsa_r, wsb_r, wna_r, wnb_r, b_ref,
             s_ref, p_ref):
        aa = _cat(a_ref)
        bb = _cat(x_ref) * d_ref[...]
        s_ref[...] = (_dotT(aa, wsa_r[...]) + _dotT(bb, wsb_r[...])
                      + b_ref[...])
        _split_store(p_ref, _dotT(aa, wna_r[...]) + _dotT(bb, wnb_r[...]))

    return pl.pallas_call(
        body,
        grid=(npad // bn,),
        in_specs=[
            _tspec(bn), _tspec(bn),
            pl.BlockSpec((bn, 1), lambda i: (i, 0)),
            _wspec(), _wspec(), _wspec(), _wspec(),
            pl.BlockSpec((1, 64), lambda i: (0, 0)),
        ],
        out_specs=[pl.BlockSpec((bn, 64), lambda i: (i, 0)), _tspec(bn)],
        out_shape=[
            jax.ShapeDtypeStruct((npad, 64), jnp.float32),
            jax.ShapeDtypeStruct((4, npad, 16), jnp.float32),
        ],
    )(a, xt, deg, wsa, wsb, wna, wnb, b.reshape(1, 64))


def _tc_inv(cnt2, sel, ctot):
    def body(c_ref, sel_ref, o_ref):
        tot = c_ref[0] + c_ref[1]
        o_ref[...] = jnp.where(sel_ref[...] > 0,
                               1.0 / jnp.maximum(tot, 1.0), tot)

    return pl.pallas_call(
        body,
        out_shape=jax.ShapeDtypeStruct((ctot // 128, 128), jnp.float32),
    )(cnt2.reshape(2, ctot // 128, 128), sel)


def _tc_mlp(ym, xnet, w1a, w1b, w1c, b1, w2, b2):
    bn = 512

    def body(y_ref, xn_ref, w1a_r, w1b_r, w1c_r, b1_r, w2_r, b2_r, o_ref):
        ya = jnp.maximum(y_ref[0], y_ref[1])
        y0 = jnp.concatenate([ya[0], ya[1]], axis=1)
        y1 = jnp.concatenate([ya[2], ya[3]], axis=1)
        y0 = jnp.where(jnp.isfinite(y0), y0, 0.0)
        y1 = jnp.where(jnp.isfinite(y1), y1, 0.0)
        h = (_dotT(y0, w1a_r[...]) + _dotT(y1, w1b_r[...])
             + _dotT(xn_ref[...], w1c_r[...]) + b1_r[...])
        h = jnp.tanh(h)
        o_ref[...] = (jnp.sum(h * w2_r[...], axis=1, keepdims=True)
                      + b2_r[0, 0])

    return pl.pallas_call(
        body,
        grid=(_NNETP // bn,),
        in_specs=[
            pl.BlockSpec((2, 4, bn, 16), lambda i: (0, 0, i, 0)),
            pl.BlockSpec((bn, 16), lambda i: (i, 0)),
            pl.BlockSpec((128, 32), lambda i: (0, 0)),
            pl.BlockSpec((128, 32), lambda i: (0, 0)),
            pl.BlockSpec((128, 16), lambda i: (0, 0)),
            pl.BlockSpec((1, 128), lambda i: (0, 0)),
            pl.BlockSpec((1, 128), lambda i: (0, 0)),
            pl.BlockSpec((1, 1), lambda i: (0, 0)),
        ],
        out_specs=pl.BlockSpec((bn, 1), lambda i: (i, 0)),
        out_shape=jax.ShapeDtypeStruct((_NNETP, 1), jnp.float32),
    )(ym, xnet, w1a, w1b, w1c, b1.reshape(1, 128), w2, b2.reshape(1, 1))


def kernel(x0, x_net, to0, to1, to2, down01_src, down01_dst, down12_src,
           down12_dst, up21_src, up21_dst, up10_src, up10_dst, conn_src,
           conn_dst, W_self_0, W_neigh_0, b_0, W_self_1, W_neigh_1, b_1,
           W_self_2, W_neigh_2, b_2, W_self_3, W_neigh_3, b_3,
           W_self_4, W_neigh_4, b_4, mlp_W1, mlp_b1, mlp_W2, mlp_b2):
    f32 = jnp.float32

    x0p = jnp.pad(x0, ((0, _N0P - _N0), (0, 0)))
    xnetp = jnp.pad(x_net, ((0, _NNETP - _NNET), (0, 0)))

    to0s, to0d = _pad_edges(to0[0], to0[1], _N0, _N0, _ep(800000))
    to1s, to1d = _pad_edges(to1[0], to1[1], _N1, _N1, _ep(200000))
    to2s, to2d = _pad_edges(to2[0], to2[1], _N2, _N2, _ep(50000))
    d01s, d01d = _pad_edges(down01_src, down01_dst, _N0, _N1, _ep(100000))
    d12s, d12d = _pad_edges(down12_src, down12_dst, _N1, _N2, _ep(25000))
    u21s, u21d = _pad_edges(up21_src, up21_dst, _N2, _N1, _ep(25000))
    u10s, u10d = _pad_edges(up10_src, up10_dst, _N1, _N0, _ep(100000))
    cns, cnd = _pad_edges(conn_src, conn_dst, _N0, _NNET, _ep(400000))

    jobs = [(to0d, 51200), (to1d, 14336), (to2d, 5120),
            (d01d, 14336), (d12d, 5120), (u21d, 14336), (u10d, 59392)]
    offs, acc_off = [], 0
    for _, ln in jobs:
        offs.append(acc_off)
        acc_off += ln
    ctot = acc_off
    cdsts = [d + o for (d, _), o in zip(jobs, offs)]
    sel = jnp.concatenate(
        [jnp.full((ln,), 1.0 if j < 5 else 0.0, f32)
         for j, (_, ln) in enumerate(jobs)]).reshape(ctot // 128, 128)

    _DEBUG_JNP_COUNTS = False
    if _DEBUG_JNP_COUNTS:
        def _cnt(d, n, npad):
            c = jax.ops.segment_sum(jnp.ones(d.shape, f32), d, num_segments=n)
            return jnp.pad(c, (0, npad - n)).reshape(npad, 1)
        inv_to0 = 1.0 / jnp.maximum(_cnt(to0[1], _N0, _N0P), 1.0)
        inv_to1 = 1.0 / jnp.maximum(_cnt(to1[1], _N1, _N1P), 1.0)
        inv_to2 = 1.0 / jnp.maximum(_cnt(to2[1], _N2, _N2P), 1.0)
        inv_d01 = 1.0 / jnp.maximum(_cnt(down01_dst, _N1, _N1P), 1.0)
        inv_d12 = 1.0 / jnp.maximum(_cnt(down12_dst, _N2, _N2P), 1.0)
        deg21 = _cnt(up21_dst, _N1, _N1P)
        deg10 = _cnt(up10_dst, _N0, _N0P)
    else:
        cnt2 = _counts_sc(cdsts, ctot)
        invdeg = _tc_inv(cnt2.reshape(2, ctot), sel, ctot).reshape(ctot)

        def seg(j, npad):
            return invdeg[offs[j]:offs[j] + npad].reshape(npad, 1)

        inv_to0, inv_to1, inv_to2 = seg(0, _N0P), seg(1, _N1P), seg(2, _N2P)
        inv_d01, inv_d12 = seg(3, _N1P), seg(4, _N2P)
        deg21, deg10 = seg(5, _N1P), seg(6, _N0P)

    def flat(p):
        return p.reshape(p.shape[0] * p.shape[1], p.shape[2])

    s0, p0 = _tc_transform(x0p, W_self_0, W_neigh_0, b_0, _N0P)
    g0 = _seg_sum_sc(flat(p0), to0s, to0d, _N0P, _N0P)
    x0t = _tc_combine(s0, g0.reshape(4, _N0P, 16), inv_to0, _N0P)

    q1 = _seg_sum_sc(flat(x0t), d01s, d01d, _N1P, _N0P)
    s1, p1 = _tc_scale_transform(q1.reshape(4, _N1P, 16), inv_d01,
                                 W_self_1, W_neigh_1, b_1, _N1P)
    g1 = _seg_sum_sc(flat(p1), to1s, to1d, _N1P, _N1P)
    x1t = _tc_combine(s1, g1.reshape(4, _N1P, 16), inv_to1, _N1P)

    q2 = _seg_sum_sc(flat(x1t), d12s, d12d, _N2P, _N1P)
    s2, p2 = _tc_scale_transform(q2.reshape(4, _N2P, 16), inv_d12,
                                 W_self_2, W_neigh_2, b_2, _N2P)
    g2 = _seg_sum_sc(flat(p2), to2s, to2d, _N2P, _N2P)
    x2t = _tc_combine(s2, g2.reshape(4, _N2P, 16), inv_to2, _N2P)

    a1 = _seg_sum_sc(flat(x2t), u21s, u21d, _N1P, _N2P)
    z1s, p3 = _tc_up_transform(a1.reshape(4, _N1P, 16), x1t, deg21,
                               W_self_3[:, :64], W_self_3[:, 64:],
                               W_neigh_3[:, :64], W_neigh_3[:, 64:],
                               b_3, _N1P)
    g3 = _seg_sum_sc(flat(p3), to1s, to1d, _N1P, _N1P)
    x1ut = _tc_combine(z1s, g3.reshape(4, _N1P, 16), inv_to1, _N1P)

    a0 = _seg_sum_sc(flat(x1ut), u10s, u10d, _N0P, _N1P)
    z0s, p4 = _tc_up_transform(a0.reshape(4, _N0P, 16), x0t, deg10,
                               W_self_4[:, :64], W_self_4[:, 64:],
                               W_neigh_4[:, :64], W_neigh_4[:, 64:],
                               b_4, _N0P)
    g4 = _seg_sum_sc(flat(p4), to0s, to0d, _N0P, _N0P)
    x0ut = _tc_combine(z0s, g4.reshape(4, _N0P, 16), inv_to0, _N0P)

    _DEBUG_JNP_READOUT = False
    if _DEBUG_JNP_READOUT:
        xu = jnp.concatenate([x0ut[q] for q in range(4)], axis=1)[:_N0]
        x1p, x2p = xu[:, :32], xu[:, 32:]
        y_max = jax.ops.segment_max(jnp.take(x1p, conn_src, axis=0),
                                    conn_dst, num_segments=_NNET)
        y_max = jnp.where(jnp.isfinite(y_max), y_max, 0.0)
        y_min = jax.ops.segment_max(jnp.take(x2p, conn_src, axis=0),
                                    conn_dst, num_segments=_NNET)
        y_min = jnp.where(jnp.isfinite(y_min), y_min, 0.0)
        xx = jnp.concatenate([y_max, y_min, x_net], axis=1)
        xx = jnp.tanh(xx @ mlp_W1.T + mlp_b1)
        return xx @ mlp_W2.T + mlp_b2
    ym = _seg_max_sc(flat(x0ut), cns, cnd)
    out = _tc_mlp(ym.reshape(2, 4, _NNETP, 16), xnetp,
                  mlp_W1[:, :32], mlp_W1[:, 32:64], mlp_W1[:, 64:],
                  mlp_b1, mlp_W2, mlp_b2)
    return out[:_NNET]


# speedup vs baseline: 3.5306x; 1.0004x over previous
import jax
import jax.numpy as jnp
from jax import lax
from jax.experimental import pallas as pl
from jax.experimental.pallas import tpu as pltpu
from jax.experimental.pallas import tpu_sc as plsc

_N0, _N1, _N2, _NNET = 50000, 12500, 3125, 20000
_N0P, _N1P, _N2P, _NNETP = 50176, 13312, 4096, 20480

_SC_PARAMS = pltpu.CompilerParams(use_tc_tiling_on_sc=False,
                                  needs_layout_passes=False)


def _mesh():
    return plsc.VectorSubcoreMesh(core_axis_name="c", subcore_axis_name="s")


def _vgather(x, idx):
    dnums = lax.GatherDimensionNumbers(offset_dims=(),
                                       collapsed_slice_dims=(0,),
                                       start_index_map=(0,))
    return lax.gather(x, idx[:, None], dnums, (1,),
                      mode=lax.GatherScatterMode.PROMISE_IN_BOUNDS)


def _pad_edges(src, dst, n_src, n_dst, ep):
    e = src.shape[0]
    extra = ep - e
    pos = jnp.arange(extra, dtype=jnp.int32)
    src_p = jnp.concatenate([src.astype(jnp.int32), pos % n_src])
    dst_p = jnp.concatenate([dst.astype(jnp.int32), n_dst + (pos % 8)])
    return src_p.reshape(ep // 128, 128), dst_p.reshape(ep // 128, 128)


def _ep(e):
    return -(-e // 32768) * 32768


def _seg_sum_sc(table2, src2d, dst2d, ndp, nsp):
    nr = src2d.shape[0]
    rpt = nr // 16
    ki = 16
    q, t = rpt // ki, rpt % ki
    zb = ndp // 16 // 64

    def body(table, src_h, dst_h, out, srcb, dstb, rows, zblk, acc, sem):
        c = lax.axis_index("c")
        s = lax.axis_index("s")
        zv = jnp.zeros((16,), jnp.float32)
        for r in range(64):
            zblk[r, pl.ds(0, 16)] = zv

        for half in range(2):
            slab = c * 2 + half

            def zloop(i, carry):
                pltpu.sync_copy(zblk,
                                acc.at[pl.ds(s * (ndp // 16) + i * 64, 64)])
                return carry

            lax.fori_loop(0, zb, zloop, 0)
            plsc.subcore_barrier()

            coff = slab * nsp

            def do_batch(row0, nb):
                pltpu.sync_copy(src_h.at[pl.ds(row0, nb)],
                                srcb.at[pl.ds(0, nb)])
                pltpu.sync_copy(dst_h.at[pl.ds(row0, nb)],
                                dstb.at[pl.ds(0, nb)])
                for j in range(nb):
                    for k in range(8):
                        srcb[j, pl.ds(k * 16, 16)] = (
                            srcb[j, pl.ds(k * 16, 16)] + coff)
                cps = [pltpu.async_copy(table.at[srcb.at[j]],
                                        rows.at[pl.ds(j * 128, 128)], sem)
                       for j in range(nb)]
                for cp in cps:
                    cp.wait()
                for j in range(nb):
                    pltpu.sync_copy(rows.at[pl.ds(j * 128, 128)],
                                    acc.at[dstb.at[j]], add=True)

            base = s * rpt

            def mloop(i, carry):
                do_batch(base + i * ki, ki)
                return carry

            lax.fori_loop(0, q, mloop, 0)
            for tt in range(t):
                do_batch(base + q * ki + tt, 1)
            plsc.subcore_barrier()

            def oloop(i, carry):
                off = s * (ndp // 16) + i * 64
                pltpu.sync_copy(acc.at[pl.ds(off, 64)],
                                out.at[pl.ds(slab * ndp + off, 64)])
                return carry

            lax.fori_loop(0, zb, oloop, 0)
            plsc.subcore_barrier()

    k = pl.kernel(
        body,
        out_type=jax.ShapeDtypeStruct((4 * ndp, 16), jnp.float32),
        mesh=_mesh(),
        compiler_params=_SC_PARAMS,
        scratch_types=[
            pltpu.VMEM((16, 128), jnp.int32),
            pltpu.VMEM((16, 128), jnp.int32),
            pltpu.VMEM((2048, 16), jnp.float32),
            pltpu.VMEM((64, 16), jnp.float32),
            pltpu.VMEM_SHARED((ndp, 16), jnp.float32),
            pltpu.SemaphoreType.DMA,
        ],
    )
    return k(table2, src2d, dst2d)


def _counts_sc(dsts, ctot):
    ct16 = ctot // 16
    zc = ct16 // 1024

    def body(*refs):
        dst_hs = refs[:len(dsts)]
        out, idxb, ones, zblk, acc, sem = refs[len(dsts):]
        c = lax.axis_index("c")
        s = lax.axis_index("s")
        wid = c * 16 + s
        ov = jnp.ones((16,), jnp.float32)
        zv = jnp.zeros((16,), jnp.float32)
        for k in range(8):
            ones[pl.ds(k * 16, 16)] = ov
        for k in range(64):
            zblk[pl.ds(k * 16, 16)] = zv

        def zloop(i, carry):
            off = pl.multiple_of(s * ct16 + i * 1024, 8)
            pltpu.sync_copy(zblk, acc.at[pl.ds(off, 1024)])
            return carry

        lax.fori_loop(0, zc, zloop, 0)
        plsc.subcore_barrier()

        for dh in dst_hs:
            rt = dh.shape[0] // 32
            qq, tt = rt // 8, rt % 8
            base = wid * rt

            def do_batch(row0, nb, dh=dh):
                pltpu.sync_copy(dh.at[pl.ds(row0, nb)], idxb.at[pl.ds(0, nb)])
                for j in range(nb):
                    pltpu.sync_copy(ones, acc.at[idxb.at[j]], add=True)

            def mloop(i, carry, base=base, do_batch=do_batch):
                do_batch(base + i * 8, 8)
                return carry

            lax.fori_loop(0, qq, mloop, 0)
            for j in range(tt):
                do_batch(base + qq * 8 + j, 1)
        plsc.subcore_barrier()

        def oloop(i, carry):
            off = pl.multiple_of(s * ct16 + i * 1024, 8)
            pltpu.sync_copy(acc.at[pl.ds(off, 1024)],
                            out.at[pl.ds(pl.multiple_of(
                                c * ctot + off, 8), 1024)])
            return carry

        lax.fori_loop(0, zc, oloop, 0)

    k = pl.kernel(
        body,
        out_type=jax.ShapeDtypeStruct((2 * ctot,), jnp.float32),
        mesh=_mesh(),
        compiler_params=_SC_PARAMS,
        scratch_types=[
            pltpu.VMEM((8, 128), jnp.int32),
            pltpu.VMEM((128,), jnp.float32),
            pltpu.VMEM((1024,), jnp.float32),
            pltpu.VMEM_SHARED((ctot,), jnp.float32),
            pltpu.SemaphoreType.DMA,
        ],
    )
    return k(*dsts)


def _seg_max_sc(table2, src2d, dst2d):
    nr = src2d.shape[0]
    nchunks = nr // 2 // 8
    rng = _NNETP // 16
    sent = rng

    def body(table, src_h, dst_h, out, sstage, dstage, csrc, cdst, csrc2,
             rows0, rows1, rows2, rows3, acc0, acc1, acc2, acc3, sem):
        c = lax.axis_index("c")
        s = lax.axis_index("s")
        lo = s * rng
        ninf = jnp.full((16,), -jnp.inf, jnp.float32)
        lane = lax.iota(jnp.int32, 16)
        accs = (acc0, acc1, acc2, acc3)
        rows = (rows0, rows1, rows2, rows3)

        def init_loop(i, carry):
            for a in accs:
                a[i, pl.ds(0, 16)] = ninf
            return carry

        lax.fori_loop(0, rng + 8, init_loop, 0)

        def flush_at(h0):
            h = pl.multiple_of(h0, 8)
            for q in range(4):
                if q:
                    for k in range(8):
                        csrc2[pl.ds(k * 16, 16)] = (
                            csrc[pl.ds(h + k * 16, 16)] + q * _N0P)
                    pltpu.async_copy(table.at[csrc2.at[pl.ds(0, 128)]],
                                     rows[q], sem).wait()
                else:
                    pltpu.async_copy(table.at[csrc.at[pl.ds(h, 128)]],
                                     rows[0], sem).wait()

            def upd(b, carry):
                dvec = cdst[pl.ds(h + b * 16, 16)]
                for ln in range(16):
                    d = dvec[ln]
                    i = b * 16 + ln
                    for a, r in zip(accs, rows):
                        a[d, pl.ds(0, 16)] = jnp.maximum(
                            a[d, pl.ds(0, 16)], r[i, pl.ds(0, 16)])
                return carry

            lax.fori_loop(0, 8, upd, 0)

        def append(state, sv, dv):
            cur, ncar, car_s, car_d = state
            m = (dv >= lo) & (dv < lo + rng)
            dl = dv - lo
            keys = lane + jnp.where(m, 0, 16)
            _, perm = plsc.sort_key_val(keys, lane)
            nv = jnp.max(plsc.all_reduce_population_count(m))
            csv = _vgather(sv, perm)
            cdl = _vgather(dl, perm)
            i1 = jnp.clip(lane - ncar, 0, 15)
            f_s = jnp.where(lane < ncar, car_s,
                            _vgather(csv, i1))
            f_d = jnp.where(lane < ncar, car_d,
                            _vgather(cdl, i1))
            total = ncar + nv
            full = total >= 16

            @pl.when(full)
            def _():
                cc = pl.multiple_of(cur, 8)
                csrc[pl.ds(cc, 16)] = f_s
                cdst[pl.ds(cc, 16)] = f_d

            i2 = jnp.clip(lane + 16 - ncar, 0, 15)
            g_s = _vgather(csv, i2)
            g_d = _vgather(cdl, i2)
            car_s = jnp.where(full, g_s, f_s)
            car_d = jnp.where(full, g_d, f_d)
            step = jnp.where(full, 16, 0)
            return cur + step, total - step, car_s, car_d

        def chunk(i, state):
            base = c * (nr // 2) + i * 8
            pltpu.sync_copy(src_h.at[pl.ds(base, 8)], sstage)
            pltpu.sync_copy(dst_h.at[pl.ds(base, 8)], dstage)
            for j in range(8):
                for k in range(8):
                    dv = dstage[j, pl.ds(k * 16, 16)]
                    sv = sstage[j, pl.ds(k * 16, 16)]
                    state = append(state, sv, dv)
            cur, ncar, car_s, car_d = state

            def wbody(h):
                flush_at(h)
                return h + 128

            h = lax.while_loop(lambda h: h + 128 <= cur, wbody, jnp.int32(0))
            hh = pl.multiple_of(h, 8)
            for k in range(8):
                csrc[pl.ds(k * 16, 16)] = csrc[pl.ds(hh + k * 16, 16)]
                cdst[pl.ds(k * 16, 16)] = cdst[pl.ds(hh + k * 16, 16)]
            return cur - h, ncar, car_s, car_d

        state0 = (jnp.int32(0), jnp.int32(0),
                  jnp.zeros((16,), jnp.int32), jnp.full((16,), sent, jnp.int32))
        cur, ncar, car_s, car_d = lax.fori_loop(0, nchunks, chunk, state0)
        cc = pl.multiple_of(cur, 8)
        csrc[pl.ds(cc, 16)] = car_s
        cdst[pl.ds(cc, 16)] = jnp.where(lane < ncar, car_d, sent)
        nedges = cur + ncar
        for k in range(8):
            v = cdst[pl.ds(k * 16, 16)]
            w = csrc[pl.ds(k * 16, 16)]
            tail = lane + k * 16 >= nedges
            cdst[pl.ds(k * 16, 16)] = jnp.where(tail, sent, v)
            csrc[pl.ds(k * 16, 16)] = jnp.where(tail, 0, w)
        flush_at(jnp.int32(0))
        for q in range(4):
            pltpu.sync_copy(
                accs[q].at[pl.ds(0, rng)],
                out.at[pl.ds((c * 4 + q) * _NNETP + lo, rng)])

    k = pl.kernel(
        body,
        out_type=jax.ShapeDtypeStruct((8 * _NNETP, 16), jnp.float32),
        mesh=_mesh(),
        compiler_params=_SC_PARAMS,
        scratch_types=[
            pltpu.VMEM((8, 128), jnp.int32),
            pltpu.VMEM((8, 128), jnp.int32),
            pltpu.VMEM((1184,), jnp.int32),
            pltpu.VMEM((1184,), jnp.int32),
            pltpu.VMEM((128,), jnp.int32),
            pltpu.VMEM((128, 16), jnp.float32),
            pltpu.VMEM((128, 16), jnp.float32),
            pltpu.VMEM((128, 16), jnp.float32),
            pltpu.VMEM((128, 16), jnp.float32),
            pltpu.VMEM((_NNETP // 16 + 8, 16), jnp.float32),
            pltpu.VMEM((_NNETP // 16 + 8, 16), jnp.float32),
            pltpu.VMEM((_NNETP // 16 + 8, 16), jnp.float32),
            pltpu.VMEM((_NNETP // 16 + 8, 16), jnp.float32),
            pltpu.SemaphoreType.DMA,
        ],
    )
    return k(table2, src2d, dst2d)


def _dotT(x, w):
    return lax.dot_general(x, w, (((1,), (1,)), ((), ())),
                           preferred_element_type=jnp.float32)


def _split_store(p_ref, p):
    for qq in range(4):
        p_ref[qq] = p[:, 16 * qq:16 * (qq + 1)]


def _cat(ref):
    return jnp.concatenate([ref[0], ref[1], ref[2], ref[3]], axis=1)


def _wspec():
    return pl.BlockSpec((64, 64), lambda i: (0, 0))


def _tspec(bn):
    return pl.BlockSpec((4, bn, 16), lambda i: (0, i, 0))


def _tc_transform(x, ws, wn, b, npad):
    bn = 512

    def body(x_ref, ws_ref, wn_ref, b_ref, s_ref, p_ref):
        xb = x_ref[...]
        s_ref[...] = _dotT(xb, ws_ref[...]) + b_ref[...]
        _split_store(p_ref, _dotT(xb, wn_ref[...]))

    return pl.pallas_call(
        body,
        grid=(npad // bn,),
        in_specs=[
            pl.BlockSpec((bn, 64), lambda i: (i, 0)),
            _wspec(), _wspec(),
            pl.BlockSpec((1, 64), lambda i: (0, 0)),
        ],
        out_specs=[pl.BlockSpec((bn, 64), lambda i: (i, 0)), _tspec(bn)],
        out_shape=[
            jax.ShapeDtypeStruct((npad, 64), jnp.float32),
            jax.ShapeDtypeStruct((4, npad, 16), jnp.float32),
        ],
    )(x, ws, wn, b.reshape(1, 64))


def _tc_combine(sarr, g, inv, npad):
    bn = 512

    def body(s_ref, g_ref, inv_ref, o_ref):
        x = jnp.tanh(s_ref[...] + _cat(g_ref) * inv_ref[...])
        _split_store(o_ref, x)

    return pl.pallas_call(
        body,
        grid=(npad // bn,),
        in_specs=[
            pl.BlockSpec((bn, 64), lambda i: (i, 0)),
            _tspec(bn),
            pl.BlockSpec((bn, 1), lambda i: (i, 0)),
        ],
        out_specs=_tspec(bn),
        out_shape=jax.ShapeDtypeStruct((4, npad, 16), jnp.float32),
    )(sarr, g, inv)


def _tc_scale_transform(q, inv, ws, wn, b, npad):
    bn = 512

    def body(q_ref, inv_ref, ws_ref, wn_ref, b_ref, s_ref, p_ref):
        x = _cat(q_ref) * inv_ref[...]
        s_ref[...] = _dotT(x, ws_ref[...]) + b_ref[...]
        _split_store(p_ref, _dotT(x, wn_ref[...]))

    return pl.pallas_call(
        body,
        grid=(npad // bn,),
        in_specs=[
            _tspec(bn),
            pl.BlockSpec((bn, 1), lambda i: (i, 0)),
            _wspec(), _wspec(),
            pl.BlockSpec((1, 64), lambda i: (0, 0)),
        ],
        out_specs=[pl.BlockSpec((bn, 64), lambda i: (i, 0)), _tspec(bn)],
        out_shape=[
            jax.ShapeDtypeStruct((npad, 64), jnp.float32),
            jax.ShapeDtypeStruct((4, npad, 16), jnp.float32),
        ],
    )(q, inv, ws, wn, b.reshape(1, 64))


def _tc_up_transform(a, xt, deg, wsa, wsb, wna, wnb, b, npad):
    bn = 512

    def body(a_ref, x_ref, d_ref, wsa_r, wsb_r, wna_r, wnb_r, b_ref,
             s_ref, p_ref):
        aa = _cat(a_ref)
        bb = _cat(x_ref) * d_ref[...]
        s_ref[...] = (_dotT(aa, wsa_r[...]) + _dotT(bb, wsb_r[...])
                      + b_ref[...])
        _split_store(p_ref, _dotT(aa, wna_r[...]) + _dotT(bb, wnb_r[...]))

    return pl.pallas_call(
        body,
        grid=(npad // bn,),
        in_specs=[
            _tspec(bn), _tspec(bn),
            pl.BlockSpec((bn, 1), lambda i: (i, 0)),
            _wspec(), _wspec(), _wspec(), _wspec(),
            pl.BlockSpec((1, 64), lambda i: (0, 0)),
        ],
        out_specs=[pl.BlockSpec((bn, 64), lambda i: (i, 0)), _tspec(bn)],
        out_shape=[
            jax.ShapeDtypeStruct((npad, 64), jnp.float32),
            jax.ShapeDtypeStruct((4, npad, 16), jnp.float32),
        ],
    )(a, xt, deg, wsa, wsb, wna, wnb, b.reshape(1, 64))


def _tc_inv(cnt2, sel, ctot):
    def body(c_ref, sel_ref, o_ref):
        tot = c_ref[0] + c_ref[1]
        o_ref[...] = jnp.where(sel_ref[...] > 0,
                               1.0 / jnp.maximum(tot, 1.0), tot)

    return pl.pallas_call(
        body,
        out_shape=jax.ShapeDtypeStruct((ctot // 128, 128), jnp.float32),
    )(cnt2.reshape(2, ctot // 128, 128), sel)


def _tc_mlp(ym, xnet, w1a, w1b, w1c, b1, w2, b2):
    bn = 512

    def body(y_ref, xn_ref, w1a_r, w1b_r, w1c_r, b1_r, w2_r, b2_r, o_ref):
        ya = jnp.maximum(y_ref[0], y_ref[1])
        y0 = jnp.concatenate([ya[0], ya[1]], axis=1)
        y1 = jnp.concatenate([ya[2], ya[3]], axis=1)
        y0 = jnp.where(jnp.isfinite(y0), y0, 0.0)
        y1 = jnp.where(jnp.isfinite(y1), y1, 0.0)
        h = (_dotT(y0, w1a_r[...]) + _dotT(y1, w1b_r[...])
             + _dotT(xn_ref[...], w1c_r[...]) + b1_r[...])
        h = jnp.tanh(h)
        o_ref[...] = (jnp.sum(h * w2_r[...], axis=1, keepdims=True)
                      + b2_r[0, 0])

    return pl.pallas_call(
        body,
        grid=(_NNETP // bn,),
        in_specs=[
            pl.BlockSpec((2, 4, bn, 16), lambda i: (0, 0, i, 0)),
            pl.BlockSpec((bn, 16), lambda i: (i, 0)),
            pl.BlockSpec((128, 32), lambda i: (0, 0)),
            pl.BlockSpec((128, 32), lambda i: (0, 0)),
            pl.BlockSpec((128, 16), lambda i: (0, 0)),
            pl.BlockSpec((1, 128), lambda i: (0, 0)),
            pl.BlockSpec((1, 128), lambda i: (0, 0)),
            pl.BlockSpec((1, 1), lambda i: (0, 0)),
        ],
        out_specs=pl.BlockSpec((bn, 1), lambda i: (i, 0)),
        out_shape=jax.ShapeDtypeStruct((_NNETP, 1), jnp.float32),
    )(ym, xnet, w1a, w1b, w1c, b1.reshape(1, 128), w2, b2.reshape(1, 1))


def kernel(x0, x_net, to0, to1, to2, down01_src, down01_dst, down12_src,
           down12_dst, up21_src, up21_dst, up10_src, up10_dst, conn_src,
           conn_dst, W_self_0, W_neigh_0, b_0, W_self_1, W_neigh_1, b_1,
           W_self_2, W_neigh_2, b_2, W_self_3, W_neigh_3, b_3,
           W_self_4, W_neigh_4, b_4, mlp_W1, mlp_b1, mlp_W2, mlp_b2):
    f32 = jnp.float32

    x0p = jnp.pad(x0, ((0, _N0P - _N0), (0, 0)))
    xnetp = jnp.pad(x_net, ((0, _NNETP - _NNET), (0, 0)))

    to0s, to0d = _pad_edges(to0[0], to0[1], _N0, _N0, _ep(800000))
    to1s, to1d = _pad_edges(to1[0], to1[1], _N1, _N1, _ep(200000))
    to2s, to2d = _pad_edges(to2[0], to2[1], _N2, _N2, _ep(50000))
    d01s, d01d = _pad_edges(down01_src, down01_dst, _N0, _N1, _ep(100000))
    d12s, d12d = _pad_edges(down12_src, down12_dst, _N1, _N2, _ep(25000))
    u21s, u21d = _pad_edges(up21_src, up21_dst, _N2, _N1, _ep(25000))
    u10s, u10d = _pad_edges(up10_src, up10_dst, _N1, _N0, _ep(100000))
    cns, cnd = _pad_edges(conn_src, conn_dst, _N0, _NNET, _ep(400000))

    jobs = [(to0d, 51200), (to1d, 14336), (to2d, 5120),
            (d01d, 14336), (d12d, 5120), (u21d, 14336), (u10d, 59392)]
    offs, acc_off = [], 0
    for _, ln in jobs:
        offs.append(acc_off)
        acc_off += ln
    ctot = acc_off
    cdsts = [d + o for (d, _), o in zip(jobs, offs)]
    sel = jnp.concatenate(
        [jnp.full((ln,), 1.0 if j < 5 else 0.0, f32)
         for j, (_, ln) in enumerate(jobs)]).reshape(ctot // 128, 128)

    cnt2 = _counts_sc(cdsts, ctot)
    invdeg = _tc_inv(cnt2.reshape(2, ctot), sel, ctot).reshape(ctot)

    def seg(j, npad):
        return invdeg[offs[j]:offs[j] + npad].reshape(npad, 1)

    inv_to0, inv_to1, inv_to2 = seg(0, _N0P), seg(1, _N1P), seg(2, _N2P)
    inv_d01, inv_d12 = seg(3, _N1P), seg(4, _N2P)
    deg21, deg10 = seg(5, _N1P), seg(6, _N0P)

    def flat(p):
        return p.reshape(p.shape[0] * p.shape[1], p.shape[2])

    s0, p0 = _tc_transform(x0p, W_self_0, W_neigh_0, b_0, _N0P)
    g0 = _seg_sum_sc(flat(p0), to0s, to0d, _N0P, _N0P)
    x0t = _tc_combine(s0, g0.reshape(4, _N0P, 16), inv_to0, _N0P)

    q1 = _seg_sum_sc(flat(x0t), d01s, d01d, _N1P, _N0P)
    s1, p1 = _tc_scale_transform(q1.reshape(4, _N1P, 16), inv_d01,
                                 W_self_1, W_neigh_1, b_1, _N1P)
    g1 = _seg_sum_sc(flat(p1), to1s, to1d, _N1P, _N1P)
    x1t = _tc_combine(s1, g1.reshape(4, _N1P, 16), inv_to1, _N1P)

    q2 = _seg_sum_sc(flat(x1t), d12s, d12d, _N2P, _N1P)
    s2, p2 = _tc_scale_transform(q2.reshape(4, _N2P, 16), inv_d12,
                                 W_self_2, W_neigh_2, b_2, _N2P)
    g2 = _seg_sum_sc(flat(p2), to2s, to2d, _N2P, _N2P)
    x2t = _tc_combine(s2, g2.reshape(4, _N2P, 16), inv_to2, _N2P)

    a1 = _seg_sum_sc(flat(x2t), u21s, u21d, _N1P, _N2P)
    z1s, p3 = _tc_up_transform(a1.reshape(4, _N1P, 16), x1t, deg21,
                               W_self_3[:, :64], W_self_3[:, 64:],
                               W_neigh_3[:, :64], W_neigh_3[:, 64:],
                               b_3, _N1P)
    g3 = _seg_sum_sc(flat(p3), to1s, to1d, _N1P, _N1P)
    x1ut = _tc_combine(z1s, g3.reshape(4, _N1P, 16), inv_to1, _N1P)

    a0 = _seg_sum_sc(flat(x1ut), u10s, u10d, _N0P, _N1P)
    z0s, p4 = _tc_up_transform(a0.reshape(4, _N0P, 16), x0t, deg10,
                               W_self_4[:, :64], W_self_4[:, 64:],
                               W_neigh_4[:, :64], W_neigh_4[:, 64:],
                               b_4, _N0P)
    g4 = _seg_sum_sc(flat(p4), to0s, to0d, _N0P, _N0P)
    x0ut = _tc_combine(z0s, g4.reshape(4, _N0P, 16), inv_to0, _N0P)

    ym = _seg_max_sc(flat(x0ut), cns, cnd)
    out = _tc_mlp(ym.reshape(2, 4, _NNETP, 16), xnetp,
                  mlp_W1[:, :32], mlp_W1[:, 32:64], mlp_W1[:, 64:],
                  mlp_b1, mlp_W2, mlp_b2)
    return out[:_NNET]
